# baseline (speedup 1.0000x reference)
import functools

import jax
import jax.numpy as jnp
from jax import lax
from jax.experimental import pallas as pl
from jax.experimental.pallas import tpu as pltpu
from jax.experimental.pallas import tpu_sc as plsc

NN = 50000
EE = 800000
HEADS = 4
CHC = 32
HID = 128

L = 16
NC = 2
NS = 16
NW = NC * NS

NP = 50176
RB = 512
GRID = NP // RB
RPT = NP // NS

B = 128
EP = 851968
ER = EP // B
NB1 = EP // NW // B
NB2 = EP // NS // B
ZR = 98
ZR1 = 392
G1 = 4
G2 = 2

_mesh = plsc.VectorSubcoreMesh(
    core_axis_name="c", subcore_axis_name="s", num_cores=NC, num_subcores=NS)


def _enc_body(x_ref, w1_ref, b1_ref, w2_ref, b2_ref, wc_ref, as_ref, ad_ref,
              h0_ref, h1_ref, h2_ref, h3_ref, ts_ref, td_ref):
    h = jnp.dot(x_ref[...], w1_ref[...], preferred_element_type=jnp.float32)
    h = jnp.maximum(h + b1_ref[...], 0.0)
    h = jnp.dot(h, w2_ref[...], preferred_element_type=jnp.float32) + b2_ref[...]
    hh = jnp.dot(h, wc_ref[...], preferred_element_type=jnp.float32)
    ts_ref[...] = jnp.dot(hh, as_ref[...], preferred_element_type=jnp.float32)
    td_ref[...] = jnp.dot(hh, ad_ref[...], preferred_element_type=jnp.float32)
    h0_ref[...] = hh[:, 0 * CHC:1 * CHC]
    h1_ref[...] = hh[:, 1 * CHC:2 * CHC]
    h2_ref[...] = hh[:, 2 * CHC:3 * CHC]
    h3_ref[...] = hh[:, 3 * CHC:4 * CHC]


def _mid_body(o_ref, bc_ref, wc_ref, as_ref, ad_ref,
              h0_ref, h1_ref, h2_ref, h3_ref, ts_ref, td_ref):
    g = jnp.concatenate(
        [jnp.maximum(o_ref[h] + bc_ref[h], 0.0) for h in range(HEADS)], axis=1)
    hh = jnp.dot(g, wc_ref[...], preferred_element_type=jnp.float32)
    ts_ref[...] = jnp.dot(hh, as_ref[...], preferred_element_type=jnp.float32)
    td_ref[...] = jnp.dot(hh, ad_ref[...], preferred_element_type=jnp.float32)
    h0_ref[...] = hh[:, 0 * CHC:1 * CHC]
    h1_ref[...] = hh[:, 1 * CHC:2 * CHC]
    h2_ref[...] = hh[:, 2 * CHC:3 * CHC]
    h3_ref[...] = hh[:, 3 * CHC:4 * CHC]


def _dec_body(o_ref, bc_ref, wd1_ref, bd1_ref, wd2_ref, bd2_ref,
              wr1_ref, br1_ref, wr2_ref, br2_ref, err_ref, rep_ref):
    h2 = jnp.concatenate(
        [o_ref[h] + bc_ref[h] for h in range(HEADS)], axis=1)
    e = jnp.maximum(
        jnp.dot(h2, wd1_ref[...], preferred_element_type=jnp.float32)
        + bd1_ref[...], 0.0)
    err_ref[...] = jnp.dot(e, wd2_ref[...],
                           preferred_element_type=jnp.float32) + bd2_ref[...]
    r = jnp.maximum(
        jnp.dot(h2, wr1_ref[...], preferred_element_type=jnp.float32)
        + br1_ref[...], 0.0)
    rep_ref[...] = jnp.dot(r, wr2_ref[...],
                           preferred_element_type=jnp.float32) + br2_ref[...]


def _den_body(dp_ref, out_ref):
    out_ref[...] = 1.0 / (dp_ref[0] + dp_ref[1] + 1e-16)


def _full(shape):
    nd = len(shape)
    return pl.BlockSpec(shape, lambda i, _nd=nd: (0,) * _nd)


def _enc(x_pad, w1, b1, w2, b2, wc, a_s, a_d):
    return pl.pallas_call(
        _enc_body,
        grid=(GRID,),
        in_specs=[
            pl.BlockSpec((RB, 8), lambda i: (i, 0)),
            _full((8, HID)), _full((1, HID)), _full((HID, HID)),
            _full((1, HID)), _full((HID, HID)), _full((HID, L)),
            _full((HID, L)),
        ],
        out_specs=[pl.BlockSpec((RB, CHC), lambda i: (i, 0))] * HEADS
        + [pl.BlockSpec((RB, L), lambda i: (i, 0))] * 2,
        out_shape=[jax.ShapeDtypeStruct((NP, CHC), jnp.float32)] * HEADS
        + [jax.ShapeDtypeStruct((NP, L), jnp.float32)] * 2,
    )(x_pad, w1, b1, w2, b2, wc, a_s, a_d)


def _mid(o, bc, wc, a_s, a_d):
    return pl.pallas_call(
        _mid_body,
        grid=(GRID,),
        in_specs=[
            pl.BlockSpec((HEADS, RB, CHC), lambda i: (0, i, 0)),
            _full((HEADS, CHC)), _full((HID, HID)), _full((HID, L)),
            _full((HID, L)),
        ],
        out_specs=[pl.BlockSpec((RB, CHC), lambda i: (i, 0))] * HEADS
        + [pl.BlockSpec((RB, L), lambda i: (i, 0))] * 2,
        out_shape=[jax.ShapeDtypeStruct((NP, CHC), jnp.float32)] * HEADS
        + [jax.ShapeDtypeStruct((NP, L), jnp.float32)] * 2,
    )(o, bc, wc, a_s, a_d)


def _dec(o, bc, wd1, bd1, wd2, bd2, wr1, br1, wr2, br2):
    return pl.pallas_call(
        _dec_body,
        grid=(GRID,),
        in_specs=[
            pl.BlockSpec((HEADS, RB, CHC), lambda i: (0, i, 0)),
            _full((HEADS, CHC)), _full((HID, 64)), _full((1, 64)),
            _full((64, 4)), _full((1, 4)), _full((HID, 64)), _full((1, 64)),
            _full((64, 1)), _full((1, 1)),
        ],
        out_specs=[pl.BlockSpec((RB, 4), lambda i: (i, 0)),
                   pl.BlockSpec((RB, 1), lambda i: (i, 0))],
        out_shape=[jax.ShapeDtypeStruct((NP, 4), jnp.float32),
                   jax.ShapeDtypeStruct((NP, 1), jnp.float32)],
    )(o, bc, wd1, bd1, wd2, bd2, wr1, br1, wr2, br2)


def _denmerge(dp):
    return pl.pallas_call(
        _den_body,
        grid=(GRID,),
        in_specs=[pl.BlockSpec((NC, RB, L), lambda i: (0, i, 0))],
        out_specs=pl.BlockSpec((RB, L), lambda i: (i, 0)),
        out_shape=jax.ShapeDtypeStruct((NP, L), jnp.float32),
    )(dp)


@functools.partial(
    pl.kernel,
    out_type=(jax.ShapeDtypeStruct((EP, L), jnp.float32),
              jax.ShapeDtypeStruct((NC, NP, L), jnp.float32)),
    mesh=_mesh,
    scratch_types=[
        pltpu.VMEM((G1, B), jnp.int32),
        pltpu.VMEM((G1, B), jnp.int32),
        pltpu.VMEM((G1 * B, L), jnp.float32),
        pltpu.VMEM((G1 * B, L), jnp.float32),
        pltpu.VMEM((G1 * B, L), jnp.float32),
        pltpu.VMEM((ZR1, L), jnp.float32),
        pltpu.SemaphoreType.DMA,
        pltpu.SemaphoreType.DMA,
        pltpu.VMEM_SHARED((NP, L), jnp.float32),
    ],
    compiler_params=pltpu.CompilerParams(use_tc_tiling_on_sc=False),
)
def _sc_pass1(src_hbm, dst_hbm, ts_hbm, td_hbm, ex_hbm, denp_hbm,
              src1, dst1, g1, g2, exb, zb, sem1, sem2, den_sh):
    cid = lax.axis_index("c")
    tid = lax.axis_index("s")
    wid = cid * NS + tid

    def _zrow(i, carry):
        zb[i, :] = jnp.zeros((L,), jnp.float32)
        return carry

    lax.fori_loop(0, ZR1, _zrow, 0)

    def _zcopy(k, carry):
        pltpu.sync_copy(zb, den_sh.at[pl.ds(tid * RPT + k * ZR1, ZR1)])
        return carry

    lax.fori_loop(0, RPT // ZR1, _zcopy, 0)
    plsc.subcore_barrier()

    row0 = wid * NB1

    def _group(g, carry):
        grow = row0 + g * G1
        pltpu.sync_copy(src_hbm.at[pl.ds(grow, G1)], src1)
        pltpu.sync_copy(dst_hbm.at[pl.ds(grow, G1)], dst1)
        waits = []
        for j in range(G1):
            waits.append(pltpu.async_copy(
                ts_hbm.at[src1.at[j]], g1.at[pl.ds(j * B, B)], sem1))
            waits.append(pltpu.async_copy(
                td_hbm.at[dst1.at[j]], g2.at[pl.ds(j * B, B)], sem2))
        for d in waits:
            d.wait()

        def _edge(e, c2):
            v = g1[e, :] + g2[e, :]
            v = jnp.maximum(v, 0.2 * v)
            exb[e, :] = jnp.exp(v)
            return c2

        lax.fori_loop(0, G1 * B, _edge, 0, unroll=4)
        for j in range(G1):
            pltpu.sync_copy(exb.at[pl.ds(j * B, B)],
                            den_sh.at[dst1.at[j]], add=True)
        pltpu.sync_copy(exb, ex_hbm.at[pl.ds(grow * B, G1 * B)])
        return carry

    lax.fori_loop(0, NB1 // G1, _group, 0)
    plsc.subcore_barrier()
    pltpu.sync_copy(den_sh.at[pl.ds(tid * RPT, RPT)],
                    denp_hbm.at[cid, pl.ds(tid * RPT, RPT)])


@functools.partial(
    pl.kernel,
    out_type=jax.ShapeDtypeStruct((HEADS, NP, CHC), jnp.float32),
    mesh=_mesh,
    scratch_types=[
        pltpu.VMEM((4, B), jnp.int32),
        pltpu.VMEM((4, B), jnp.int32),
        pltpu.VMEM((B, L), jnp.float32),
        pltpu.VMEM((B, L), jnp.float32),
        pltpu.VMEM((B, L), jnp.float32),
        pltpu.VMEM((B, L), jnp.float32),
        pltpu.VMEM((B, CHC), jnp.float32),
        pltpu.VMEM((B, CHC), jnp.float32),
        pltpu.VMEM((B, CHC), jnp.float32),
        pltpu.VMEM((B, CHC), jnp.float32),
        pltpu.VMEM((ZR, CHC), jnp.float32),
        pltpu.SemaphoreType.DMA,
        pltpu.SemaphoreType.DMA,
        pltpu.SemaphoreType.DMA,
        pltpu.SemaphoreType.DMA,
        pltpu.VMEM_SHARED((NP, CHC), jnp.float32),
    ],
    compiler_params=pltpu.CompilerParams(use_tc_tiling_on_sc=False),
)
def _sc_pass2(src_hbm, dst_hbm, ex_hbm, denr_hbm, hh0, hh1, hh2, hh3,
              out_hbm, src_i, dst_i, exb0, exb1, dg0, dg1, hg0, hg1,
              sb0, sb1, zb, semg0, semg1, sems0, sems1, acc_sh):
    cid = lax.axis_index("c")
    tid = lax.axis_index("s")
    exb = (exb0, exb1)
    dg = (dg0, dg1)
    hg = (hg0, hg1)
    sb = (sb0, sb1)
    semg = (semg0, semg1)
    sems = (sems0, sems1)

    def _zrow(i, carry):
        zb[i, pl.ds(0, L)] = jnp.zeros((L,), jnp.float32)
        zb[i, pl.ds(L, L)] = jnp.zeros((L,), jnp.float32)
        return carry

    lax.fori_loop(0, ZR, _zrow, 0)

    def _sweep(hh_ref, slot):
        def _zcopy(k, carry):
            pltpu.sync_copy(zb, acc_sh.at[pl.ds(tid * RPT + k * ZR, ZR)])
            return carry

        lax.fori_loop(0, RPT // ZR, _zcopy, 0)
        plsc.subcore_barrier()
        row0 = tid * NB2

        def _issue(g, p, k):
            pltpu.async_copy(
                ex_hbm.at[pl.ds((row0 + g) * B, B)], exb[p], semg[p])
            pltpu.async_copy(denr_hbm.at[dst_i.at[k]], dg[p], semg[p])
            pltpu.async_copy(hh_ref.at[src_i.at[k]], hg[p], semg[p])

        def _drain_g(p, k):
            pltpu.make_async_copy(
                ex_hbm.at[pl.ds(row0 * B, B)], exb[p], semg[p]).wait()
            pltpu.make_async_copy(
                denr_hbm.at[dst_i.at[k]], dg[p], semg[p]).wait()
            pltpu.make_async_copy(
                hh_ref.at[src_i.at[k]], hg[p], semg[p]).wait()

        def _drain_s(p, k):
            pltpu.make_async_copy(
                sb[p], acc_sh.at[dst_i.at[k]], sems[p]).wait()

        for g0 in range(2):
            pltpu.sync_copy(src_hbm.at[row0 + g0], src_i.at[g0])
            pltpu.sync_copy(dst_hbm.at[row0 + g0], dst_i.at[g0])
            _issue(g0, g0, g0)

        def _quad(q, carry):
            for k in range(4):
                g = 4 * q + k
                p = k % 2

                @pl.when(g >= 2)
                def _():
                    _drain_s(p, (k + 2) % 4)

                _drain_g(p, k)

                def _edge(e, c2):
                    cv = exb[p][e, :] * dg[p][e, :]
                    c = cv[slot]
                    sb[p][e, pl.ds(0, L)] = hg[p][e, pl.ds(0, L)] * c
                    sb[p][e, pl.ds(L, L)] = hg[p][e, pl.ds(L, L)] * c
                    return c2

                lax.fori_loop(0, B, _edge, 0, unroll=8)
                pltpu.async_copy(sb[p], acc_sh.at[dst_i.at[k]],
                                 sems[p], add=True)

                @pl.when(g + 2 < NB2)
                def _():
                    k2 = (k + 2) % 4
                    pltpu.sync_copy(src_hbm.at[row0 + g + 2], src_i.at[k2])
                    pltpu.sync_copy(dst_hbm.at[row0 + g + 2], dst_i.at[k2])
                    _issue(g + 2, p, k2)
            return carry

        lax.fori_loop(0, NB2 // 4, _quad, 0)
        _drain_s(0, (NB2 - 2) % 4)
        _drain_s(1, (NB2 - 1) % 4)
        plsc.subcore_barrier()
        pltpu.sync_copy(acc_sh.at[pl.ds(tid * RPT, RPT)],
                        out_hbm.at[slot, pl.ds(tid * RPT, RPT)])
        plsc.subcore_barrier()

    @pl.when(cid == 0)
    def _():
        _sweep(hh0, 0)
        _sweep(hh1, 1)

    @pl.when(cid == 1)
    def _():
        _sweep(hh2, 2)
        _sweep(hh3, 3)


def _attn_mat(a):
    m = jnp.zeros((HID, L), jnp.float32)
    for h in range(HEADS):
        m = m.at[h * CHC:(h + 1) * CHC, h].set(a[h])
    return m


def kernel(x, edge_index, W1e, b1e, W2e, b2e, Wc1, as1, ad1, bc1,
           Wc2, as2, ad2, bc2, Wd1, bd1, Wd2, bd2, Wr1, br1, Wr2, br2):
    x_pad = jnp.zeros((NP, 8), jnp.float32).at[:NN].set(x)
    loop_idx = jnp.arange(NN, dtype=jnp.int32)
    pad_idx = jnp.full((EP - EE - NN,), NN, jnp.int32)
    src = jnp.concatenate(
        [edge_index[0].astype(jnp.int32), loop_idx, pad_idx]).reshape(ER, B)
    dst = jnp.concatenate(
        [edge_index[1].astype(jnp.int32), loop_idx, pad_idx]).reshape(ER, B)

    h0, h1, h2, h3, ts, td = _enc(
        x_pad, W1e, b1e.reshape(1, HID), W2e, b2e.reshape(1, HID),
        Wc1, _attn_mat(as1), _attn_mat(ad1))
    ex1, denp1 = _sc_pass1(src, dst, ts, td)
    denr1 = _denmerge(denp1)
    out1 = _sc_pass2(src, dst, ex1, denr1, h0, h1, h2, h3)

    h0, h1, h2, h3, ts, td = _mid(
        out1, bc1.reshape(HEADS, CHC), Wc2, _attn_mat(as2), _attn_mat(ad2))
    ex2, denp2 = _sc_pass1(src, dst, ts, td)
    denr2 = _denmerge(denp2)
    out2 = _sc_pass2(src, dst, ex2, denr2, h0, h1, h2, h3)

    err, rep = _dec(
        out2, bc2.reshape(HEADS, CHC), Wd1, bd1.reshape(1, 64),
        Wd2, bd2.reshape(1, 4), Wr1, br1.reshape(1, 64),
        Wr2, br2.reshape(1, 1))
    return (err[:NN], rep[:NN])


# speedup vs baseline: 1.3488x; 1.3488x over previous
import functools

import jax
import jax.numpy as jnp
from jax import lax
from jax.experimental import pallas as pl
from jax.experimental.pallas import tpu as pltpu
from jax.experimental.pallas import tpu_sc as plsc

NN = 50000
EE = 800000
HEADS = 4
CHC = 32
HID = 128

L = 16
NC = 2
NS = 16
NW = NC * NS

NP = 50176
RB = 512
GRID = NP // RB
RPT = NP // NS

B = 128
EP = 851968
ER = EP // B
NB1 = EP // NW // B
NB2 = EP // NS // B
ZR = 98
ZR1 = 392
G1 = 4
G2 = 2

_mesh = plsc.VectorSubcoreMesh(
    core_axis_name="c", subcore_axis_name="s", num_cores=NC, num_subcores=NS)
def _lane_bcast(v, idx):
    return lax.gather(
        v, idx[:, None],
        dimension_numbers=lax.GatherDimensionNumbers(
            offset_dims=(), collapsed_slice_dims=(0,), start_index_map=(0,)),
        slice_sizes=(1,),
        mode=lax.GatherScatterMode.PROMISE_IN_BOUNDS)


def _enc_body(x_ref, w1_ref, b1_ref, w2_ref, b2_ref, wc_ref, as_ref, ad_ref,
              h0_ref, h1_ref, h2_ref, h3_ref, ts_ref, td_ref):
    h = jnp.dot(x_ref[...], w1_ref[...], preferred_element_type=jnp.float32)
    h = jnp.maximum(h + b1_ref[...], 0.0)
    h = jnp.dot(h, w2_ref[...], preferred_element_type=jnp.float32) + b2_ref[...]
    hh = jnp.dot(h, wc_ref[...], preferred_element_type=jnp.float32)
    ts_ref[...] = jnp.dot(hh, as_ref[...], preferred_element_type=jnp.float32)
    td_ref[...] = jnp.dot(hh, ad_ref[...], preferred_element_type=jnp.float32)
    h0_ref[...] = hh[:, 0 * CHC:1 * CHC]
    h1_ref[...] = hh[:, 1 * CHC:2 * CHC]
    h2_ref[...] = hh[:, 2 * CHC:3 * CHC]
    h3_ref[...] = hh[:, 3 * CHC:4 * CHC]


def _mid_body(o_ref, bc_ref, wc_ref, as_ref, ad_ref,
              h0_ref, h1_ref, h2_ref, h3_ref, ts_ref, td_ref):
    g = jnp.concatenate(
        [jnp.maximum(o_ref[h] + bc_ref[h], 0.0) for h in range(HEADS)], axis=1)
    hh = jnp.dot(g, wc_ref[...], preferred_element_type=jnp.float32)
    ts_ref[...] = jnp.dot(hh, as_ref[...], preferred_element_type=jnp.float32)
    td_ref[...] = jnp.dot(hh, ad_ref[...], preferred_element_type=jnp.float32)
    h0_ref[...] = hh[:, 0 * CHC:1 * CHC]
    h1_ref[...] = hh[:, 1 * CHC:2 * CHC]
    h2_ref[...] = hh[:, 2 * CHC:3 * CHC]
    h3_ref[...] = hh[:, 3 * CHC:4 * CHC]


def _dec_body(o_ref, bc_ref, wd1_ref, bd1_ref, wd2_ref, bd2_ref,
              wr1_ref, br1_ref, wr2_ref, br2_ref, err_ref, rep_ref):
    h2 = jnp.concatenate(
        [o_ref[h] + bc_ref[h] for h in range(HEADS)], axis=1)
    e = jnp.maximum(
        jnp.dot(h2, wd1_ref[...], preferred_element_type=jnp.float32)
        + bd1_ref[...], 0.0)
    err_ref[...] = jnp.dot(e, wd2_ref[...],
                           preferred_element_type=jnp.float32) + bd2_ref[...]
    r = jnp.maximum(
        jnp.dot(h2, wr1_ref[...], preferred_element_type=jnp.float32)
        + br1_ref[...], 0.0)
    rep_ref[...] = jnp.dot(r, wr2_ref[...],
                           preferred_element_type=jnp.float32) + br2_ref[...]


def _den_body(dp_ref, out_ref):
    out_ref[...] = 1.0 / (dp_ref[0] + dp_ref[1] + 1e-16)


def _full(shape):
    nd = len(shape)
    return pl.BlockSpec(shape, lambda i, _nd=nd: (0,) * _nd)


def _enc(x_pad, w1, b1, w2, b2, wc, a_s, a_d):
    return pl.pallas_call(
        _enc_body,
        grid=(GRID,),
        in_specs=[
            pl.BlockSpec((RB, 8), lambda i: (i, 0)),
            _full((8, HID)), _full((1, HID)), _full((HID, HID)),
            _full((1, HID)), _full((HID, HID)), _full((HID, L)),
            _full((HID, L)),
        ],
        out_specs=[pl.BlockSpec((RB, CHC), lambda i: (i, 0))] * HEADS
        + [pl.BlockSpec((RB, L), lambda i: (i, 0))] * 2,
        out_shape=[jax.ShapeDtypeStruct((NP, CHC), jnp.float32)] * HEADS
        + [jax.ShapeDtypeStruct((NP, L), jnp.float32)] * 2,
    )(x_pad, w1, b1, w2, b2, wc, a_s, a_d)


def _mid(o, bc, wc, a_s, a_d):
    return pl.pallas_call(
        _mid_body,
        grid=(GRID,),
        in_specs=[
            pl.BlockSpec((HEADS, RB, CHC), lambda i: (0, i, 0)),
            _full((HEADS, CHC)), _full((HID, HID)), _full((HID, L)),
            _full((HID, L)),
        ],
        out_specs=[pl.BlockSpec((RB, CHC), lambda i: (i, 0))] * HEADS
        + [pl.BlockSpec((RB, L), lambda i: (i, 0))] * 2,
        out_shape=[jax.ShapeDtypeStruct((NP, CHC), jnp.float32)] * HEADS
        + [jax.ShapeDtypeStruct((NP, L), jnp.float32)] * 2,
    )(o, bc, wc, a_s, a_d)


def _dec(o, bc, wd1, bd1, wd2, bd2, wr1, br1, wr2, br2):
    return pl.pallas_call(
        _dec_body,
        grid=(GRID,),
        in_specs=[
            pl.BlockSpec((HEADS, RB, CHC), lambda i: (0, i, 0)),
            _full((HEADS, CHC)), _full((HID, 64)), _full((1, 64)),
            _full((64, 4)), _full((1, 4)), _full((HID, 64)), _full((1, 64)),
            _full((64, 1)), _full((1, 1)),
        ],
        out_specs=[pl.BlockSpec((RB, 4), lambda i: (i, 0)),
                   pl.BlockSpec((RB, 1), lambda i: (i, 0))],
        out_shape=[jax.ShapeDtypeStruct((NP, 4), jnp.float32),
                   jax.ShapeDtypeStruct((NP, 1), jnp.float32)],
    )(o, bc, wd1, bd1, wd2, bd2, wr1, br1, wr2, br2)


def _denmerge(dp):
    return pl.pallas_call(
        _den_body,
        grid=(GRID,),
        in_specs=[pl.BlockSpec((NC, RB, L), lambda i: (0, i, 0))],
        out_specs=pl.BlockSpec((RB, L), lambda i: (i, 0)),
        out_shape=jax.ShapeDtypeStruct((NP, L), jnp.float32),
    )(dp)


@functools.partial(
    pl.kernel,
    out_type=(jax.ShapeDtypeStruct((EP, L), jnp.float32),
              jax.ShapeDtypeStruct((NC, NP, L), jnp.float32)),
    mesh=_mesh,
    scratch_types=[
        pltpu.VMEM((G1, B), jnp.int32),
        pltpu.VMEM((G1, B), jnp.int32),
        pltpu.VMEM((G1 * B, L), jnp.float32),
        pltpu.VMEM((G1 * B, L), jnp.float32),
        pltpu.VMEM((G1 * B, L), jnp.float32),
        pltpu.VMEM((ZR1, L), jnp.float32),
        pltpu.SemaphoreType.DMA,
        pltpu.SemaphoreType.DMA,
        pltpu.VMEM_SHARED((NP, L), jnp.float32),
    ],
    compiler_params=pltpu.CompilerParams(use_tc_tiling_on_sc=False),
)
def _sc_pass1(src_hbm, dst_hbm, ts_hbm, td_hbm, ex_hbm, denp_hbm,
              src1, dst1, g1, g2, exb, zb, sem1, sem2, den_sh):
    cid = lax.axis_index("c")
    tid = lax.axis_index("s")
    wid = cid * NS + tid

    def _zrow(i, carry):
        zb[i, :] = jnp.zeros((L,), jnp.float32)
        return carry

    lax.fori_loop(0, ZR1, _zrow, 0)

    def _zcopy(k, carry):
        pltpu.sync_copy(zb, den_sh.at[pl.ds(tid * RPT + k * ZR1, ZR1)])
        return carry

    lax.fori_loop(0, RPT // ZR1, _zcopy, 0)
    plsc.subcore_barrier()

    row0 = wid * NB1

    def _group(g, carry):
        grow = row0 + g * G1
        pltpu.sync_copy(src_hbm.at[pl.ds(grow, G1)], src1)
        pltpu.sync_copy(dst_hbm.at[pl.ds(grow, G1)], dst1)
        waits = []
        for j in range(G1):
            waits.append(pltpu.async_copy(
                ts_hbm.at[src1.at[j]], g1.at[pl.ds(j * B, B)], sem1))
            waits.append(pltpu.async_copy(
                td_hbm.at[dst1.at[j]], g2.at[pl.ds(j * B, B)], sem2))
        for d in waits:
            d.wait()

        def _edge(e, c2):
            v = g1[e, :] + g2[e, :]
            v = jnp.maximum(v, 0.2 * v)
            exb[e, :] = jnp.exp(v)
            return c2

        lax.fori_loop(0, G1 * B, _edge, 0, unroll=4)
        for j in range(G1):
            pltpu.sync_copy(exb.at[pl.ds(j * B, B)],
                            den_sh.at[dst1.at[j]], add=True)
        pltpu.sync_copy(exb, ex_hbm.at[pl.ds(grow * B, G1 * B)])
        return carry

    lax.fori_loop(0, NB1 // G1, _group, 0)
    plsc.subcore_barrier()
    pltpu.sync_copy(den_sh.at[pl.ds(tid * RPT, RPT)],
                    denp_hbm.at[cid, pl.ds(tid * RPT, RPT)])


@functools.partial(
    pl.kernel,
    out_type=jax.ShapeDtypeStruct((HEADS, NP, CHC), jnp.float32),
    mesh=_mesh,
    scratch_types=[
        pltpu.VMEM((4, B), jnp.int32),
        pltpu.VMEM((4, B), jnp.int32),
        pltpu.VMEM((B, L), jnp.float32),
        pltpu.VMEM((B, L), jnp.float32),
        pltpu.VMEM((B, L), jnp.float32),
        pltpu.VMEM((B, L), jnp.float32),
        pltpu.VMEM((B, CHC), jnp.float32),
        pltpu.VMEM((B, CHC), jnp.float32),
        pltpu.VMEM((B, CHC), jnp.float32),
        pltpu.VMEM((B, CHC), jnp.float32),
        pltpu.VMEM((ZR, CHC), jnp.float32),
        pltpu.SemaphoreType.DMA,
        pltpu.SemaphoreType.DMA,
        pltpu.SemaphoreType.DMA,
        pltpu.SemaphoreType.DMA,
        pltpu.VMEM_SHARED((NP, CHC), jnp.float32),
    ],
    compiler_params=pltpu.CompilerParams(
        use_tc_tiling_on_sc=False, needs_layout_passes=False),
)
def _sc_pass2(src_hbm, dst_hbm, ex_hbm, denr_hbm, hh0, hh1, hh2, hh3,
              out_hbm, src_i, dst_i, exb0, exb1, dg0, dg1, hg0, hg1,
              sb0, sb1, zb, semg0, semg1, sems0, sems1, acc_sh):
    cid = lax.axis_index("c")
    tid = lax.axis_index("s")
    exb = (exb0, exb1)
    dg = (dg0, dg1)
    hg = (hg0, hg1)
    sb = (sb0, sb1)
    semg = (semg0, semg1)
    sems = (sems0, sems1)

    def _zrow(i, carry):
        zb[i, pl.ds(0, L)] = jnp.zeros((L,), jnp.float32)
        zb[i, pl.ds(L, L)] = jnp.zeros((L,), jnp.float32)
        return carry

    lax.fori_loop(0, ZR, _zrow, 0)
    iota = lax.iota(jnp.int32, L)
    jfull = [jnp.full((L,), j, jnp.int32) for j in range(L)]

    def _sweep(hh_ref, slot):
        cslot = jnp.full((L,), slot, jnp.int32)

        def _zcopy(k, carry):
            pltpu.sync_copy(zb, acc_sh.at[pl.ds(tid * RPT + k * ZR, ZR)])
            return carry

        lax.fori_loop(0, RPT // ZR, _zcopy, 0)
        plsc.subcore_barrier()
        row0 = tid * NB2

        def _issue(g, p, k):
            pltpu.async_copy(
                ex_hbm.at[pl.ds((row0 + g) * B, B)], exb[p], semg[p])
            pltpu.async_copy(denr_hbm.at[dst_i.at[k]], dg[p], semg[p])
            pltpu.async_copy(hh_ref.at[src_i.at[k]], hg[p], semg[p])

        def _drain_g(p, k):
            pltpu.make_async_copy(
                ex_hbm.at[pl.ds(row0 * B, B)], exb[p], semg[p]).wait()
            pltpu.make_async_copy(
                denr_hbm.at[dst_i.at[k]], dg[p], semg[p]).wait()
            pltpu.make_async_copy(
                hh_ref.at[src_i.at[k]], hg[p], semg[p]).wait()

        def _drain_s(p, k):
            pltpu.make_async_copy(
                sb[p], acc_sh.at[dst_i.at[k]], sems[p]).wait()

        for g0 in range(2):
            pltpu.sync_copy(src_hbm.at[row0 + g0], src_i.at[g0])
            pltpu.sync_copy(dst_hbm.at[row0 + g0], dst_i.at[g0])
            _issue(g0, g0, g0)

        def _quad(q, carry):
            for k in range(4):
                g = 4 * q + k
                p = k % 2

                @pl.when(g >= 2)
                def _():
                    _drain_s(p, (k + 2) % 4)

                _drain_g(p, k)

                def _grp(i, c2):
                    ridx = i * L + iota
                    ev = plsc.load_gather(exb[p], [ridx, cslot])
                    dv = plsc.load_gather(dg[p], [ridx, cslot])
                    cv = ev * dv
                    for j in range(L):
                        e = i * L + j
                        cj = _lane_bcast(cv, jfull[j])
                        sb[p][e, pl.ds(0, L)] = hg[p][e, pl.ds(0, L)] * cj
                        sb[p][e, pl.ds(L, L)] = hg[p][e, pl.ds(L, L)] * cj
                    return c2

                lax.fori_loop(0, B // L, _grp, 0)
                pltpu.async_copy(sb[p], acc_sh.at[dst_i.at[k]],
                                 sems[p], add=True)

                @pl.when(g + 2 < NB2)
                def _():
                    k2 = (k + 2) % 4
                    pltpu.sync_copy(src_hbm.at[row0 + g + 2], src_i.at[k2])
                    pltpu.sync_copy(dst_hbm.at[row0 + g + 2], dst_i.at[k2])
                    _issue(g + 2, p, k2)
            return carry

        lax.fori_loop(0, NB2 // 4, _quad, 0)
        _drain_s(0, (NB2 - 2) % 4)
        _drain_s(1, (NB2 - 1) % 4)
        plsc.subcore_barrier()
        pltpu.sync_copy(acc_sh.at[pl.ds(tid * RPT, RPT)],
                        out_hbm.at[slot, pl.ds(tid * RPT, RPT)])
        plsc.subcore_barrier()

    @pl.when(cid == 0)
    def _():
        _sweep(hh0, 0)
        _sweep(hh1, 1)

    @pl.when(cid == 1)
    def _():
        _sweep(hh2, 2)
        _sweep(hh3, 3)


def _attn_mat(a):
    m = jnp.zeros((HID, L), jnp.float32)
    for h in range(HEADS):
        m = m.at[h * CHC:(h + 1) * CHC, h].set(a[h])
    return m


def kernel(x, edge_index, W1e, b1e, W2e, b2e, Wc1, as1, ad1, bc1,
           Wc2, as2, ad2, bc2, Wd1, bd1, Wd2, bd2, Wr1, br1, Wr2, br2):
    x_pad = jnp.zeros((NP, 8), jnp.float32).at[:NN].set(x)
    loop_idx = jnp.arange(NN, dtype=jnp.int32)
    pad_idx = jnp.full((EP - EE - NN,), NN, jnp.int32)
    src = jnp.concatenate(
        [edge_index[0].astype(jnp.int32), loop_idx, pad_idx]).reshape(ER, B)
    dst = jnp.concatenate(
        [edge_index[1].astype(jnp.int32), loop_idx, pad_idx]).reshape(ER, B)

    h0, h1, h2, h3, ts, td = _enc(
        x_pad, W1e, b1e.reshape(1, HID), W2e, b2e.reshape(1, HID),
        Wc1, _attn_mat(as1), _attn_mat(ad1))
    ex1, denp1 = _sc_pass1(src, dst, ts, td)
    denr1 = _denmerge(denp1)
    out1 = _sc_pass2(src, dst, ex1, denr1, h0, h1, h2, h3)

    h0, h1, h2, h3, ts, td = _mid(
        out1, bc1.reshape(HEADS, CHC), Wc2, _attn_mat(as2), _attn_mat(ad2))
    ex2, denp2 = _sc_pass1(src, dst, ts, td)
    denr2 = _denmerge(denp2)
    out2 = _sc_pass2(src, dst, ex2, denr2, h0, h1, h2, h3)

    err, rep = _dec(
        out2, bc2.reshape(HEADS, CHC), Wd1, bd1.reshape(1, 64),
        Wd2, bd2.reshape(1, 4), Wr1, br1.reshape(1, 64),
        Wr2, br2.reshape(1, 1))
    return (err[:NN], rep[:NN])


# speedup vs baseline: 1.7134x; 1.2703x over previous
import functools

import jax
import jax.numpy as jnp
from jax import lax
from jax.experimental import pallas as pl
from jax.experimental.pallas import tpu as pltpu
from jax.experimental.pallas import tpu_sc as plsc

NN = 50000
EE = 800000
HEADS = 4
CHC = 32
HID = 128

L = 16
NC = 2
NS = 16
NW = NC * NS

NP = 50176
RB = 512
GRID = NP // RB
RPT = NP // NS

B = 128
EP = 851968
ER = EP // B
NB1 = EP // NW // B
NB2 = EP // NS // B
ZR = 98
ZR1 = 392
G1 = 4
G2 = 2

_mesh = plsc.VectorSubcoreMesh(
    core_axis_name="c", subcore_axis_name="s", num_cores=NC, num_subcores=NS)
def _lane_bcast(v, idx):
    return lax.gather(
        v, idx[:, None],
        dimension_numbers=lax.GatherDimensionNumbers(
            offset_dims=(), collapsed_slice_dims=(0,), start_index_map=(0,)),
        slice_sizes=(1,),
        mode=lax.GatherScatterMode.PROMISE_IN_BOUNDS)


def _enc_body(x_ref, w1_ref, b1_ref, w2_ref, b2_ref, wc_ref, as_ref, ad_ref,
              h0_ref, h1_ref, h2_ref, h3_ref, ts_ref, td_ref):
    h = jnp.dot(x_ref[...], w1_ref[...], preferred_element_type=jnp.float32)
    h = jnp.maximum(h + b1_ref[...], 0.0)
    h = jnp.dot(h, w2_ref[...], preferred_element_type=jnp.float32) + b2_ref[...]
    hh = jnp.dot(h, wc_ref[...], preferred_element_type=jnp.float32)
    ts_ref[...] = jnp.dot(hh, as_ref[...], preferred_element_type=jnp.float32)
    td_ref[...] = jnp.dot(hh, ad_ref[...], preferred_element_type=jnp.float32)
    h0_ref[...] = hh[:, 0 * CHC:1 * CHC]
    h1_ref[...] = hh[:, 1 * CHC:2 * CHC]
    h2_ref[...] = hh[:, 2 * CHC:3 * CHC]
    h3_ref[...] = hh[:, 3 * CHC:4 * CHC]


def _mid_body(o_ref, bc_ref, wc_ref, as_ref, ad_ref,
              h0_ref, h1_ref, h2_ref, h3_ref, ts_ref, td_ref):
    g = jnp.concatenate(
        [jnp.maximum(o_ref[h] + bc_ref[h], 0.0) for h in range(HEADS)], axis=1)
    hh = jnp.dot(g, wc_ref[...], preferred_element_type=jnp.float32)
    ts_ref[...] = jnp.dot(hh, as_ref[...], preferred_element_type=jnp.float32)
    td_ref[...] = jnp.dot(hh, ad_ref[...], preferred_element_type=jnp.float32)
    h0_ref[...] = hh[:, 0 * CHC:1 * CHC]
    h1_ref[...] = hh[:, 1 * CHC:2 * CHC]
    h2_ref[...] = hh[:, 2 * CHC:3 * CHC]
    h3_ref[...] = hh[:, 3 * CHC:4 * CHC]


def _dec_body(o_ref, bc_ref, wd1_ref, bd1_ref, wd2_ref, bd2_ref,
              wr1_ref, br1_ref, wr2_ref, br2_ref, err_ref, rep_ref):
    h2 = jnp.concatenate(
        [o_ref[h] + bc_ref[h] for h in range(HEADS)], axis=1)
    e = jnp.maximum(
        jnp.dot(h2, wd1_ref[...], preferred_element_type=jnp.float32)
        + bd1_ref[...], 0.0)
    err_ref[...] = jnp.dot(e, wd2_ref[...],
                           preferred_element_type=jnp.float32) + bd2_ref[...]
    r = jnp.maximum(
        jnp.dot(h2, wr1_ref[...], preferred_element_type=jnp.float32)
        + br1_ref[...], 0.0)
    rep_ref[...] = jnp.dot(r, wr2_ref[...],
                           preferred_element_type=jnp.float32) + br2_ref[...]


def _den_body(dp_ref, out_ref):
    out_ref[...] = 1.0 / (dp_ref[0] + dp_ref[1] + 1e-16)


def _full(shape):
    nd = len(shape)
    return pl.BlockSpec(shape, lambda i, _nd=nd: (0,) * _nd)


def _enc(x_pad, w1, b1, w2, b2, wc, a_s, a_d):
    return pl.pallas_call(
        _enc_body,
        grid=(GRID,),
        in_specs=[
            pl.BlockSpec((RB, 8), lambda i: (i, 0)),
            _full((8, HID)), _full((1, HID)), _full((HID, HID)),
            _full((1, HID)), _full((HID, HID)), _full((HID, L)),
            _full((HID, L)),
        ],
        out_specs=[pl.BlockSpec((RB, CHC), lambda i: (i, 0))] * HEADS
        + [pl.BlockSpec((RB, L), lambda i: (i, 0))] * 2,
        out_shape=[jax.ShapeDtypeStruct((NP, CHC), jnp.float32)] * HEADS
        + [jax.ShapeDtypeStruct((NP, L), jnp.float32)] * 2,
    )(x_pad, w1, b1, w2, b2, wc, a_s, a_d)


def _mid(o, bc, wc, a_s, a_d):
    return pl.pallas_call(
        _mid_body,
        grid=(GRID,),
        in_specs=[
            pl.BlockSpec((HEADS, RB, CHC), lambda i: (0, i, 0)),
            _full((HEADS, CHC)), _full((HID, HID)), _full((HID, L)),
            _full((HID, L)),
        ],
        out_specs=[pl.BlockSpec((RB, CHC), lambda i: (i, 0))] * HEADS
        + [pl.BlockSpec((RB, L), lambda i: (i, 0))] * 2,
        out_shape=[jax.ShapeDtypeStruct((NP, CHC), jnp.float32)] * HEADS
        + [jax.ShapeDtypeStruct((NP, L), jnp.float32)] * 2,
    )(o, bc, wc, a_s, a_d)


def _dec(o, bc, wd1, bd1, wd2, bd2, wr1, br1, wr2, br2):
    return pl.pallas_call(
        _dec_body,
        grid=(GRID,),
        in_specs=[
            pl.BlockSpec((HEADS, RB, CHC), lambda i: (0, i, 0)),
            _full((HEADS, CHC)), _full((HID, 64)), _full((1, 64)),
            _full((64, 4)), _full((1, 4)), _full((HID, 64)), _full((1, 64)),
            _full((64, 1)), _full((1, 1)),
        ],
        out_specs=[pl.BlockSpec((RB, 4), lambda i: (i, 0)),
                   pl.BlockSpec((RB, 1), lambda i: (i, 0))],
        out_shape=[jax.ShapeDtypeStruct((NP, 4), jnp.float32),
                   jax.ShapeDtypeStruct((NP, 1), jnp.float32)],
    )(o, bc, wd1, bd1, wd2, bd2, wr1, br1, wr2, br2)


def _denmerge(dp):
    return pl.pallas_call(
        _den_body,
        grid=(GRID,),
        in_specs=[pl.BlockSpec((NC, RB, L), lambda i: (0, i, 0))],
        out_specs=pl.BlockSpec((RB, L), lambda i: (i, 0)),
        out_shape=jax.ShapeDtypeStruct((NP, L), jnp.float32),
    )(dp)


@functools.partial(
    pl.kernel,
    out_type=(jax.ShapeDtypeStruct((EP, L), jnp.float32),
              jax.ShapeDtypeStruct((NC, NP, L), jnp.float32)),
    mesh=_mesh,
    scratch_types=[
        pltpu.VMEM((G1, B), jnp.int32),
        pltpu.VMEM((G1, B), jnp.int32),
        pltpu.VMEM((G1 * B, L), jnp.float32),
        pltpu.VMEM((G1 * B, L), jnp.float32),
        pltpu.VMEM((G1 * B, L), jnp.float32),
        pltpu.VMEM((ZR1, L), jnp.float32),
        pltpu.SemaphoreType.DMA,
        pltpu.SemaphoreType.DMA,
        pltpu.VMEM_SHARED((NP, L), jnp.float32),
    ],
    compiler_params=pltpu.CompilerParams(use_tc_tiling_on_sc=False),
)
def _sc_pass1(src_hbm, dst_hbm, ts_hbm, td_hbm, ex_hbm, denp_hbm,
              src1, dst1, g1, g2, exb, zb, sem1, sem2, den_sh):
    cid = lax.axis_index("c")
    tid = lax.axis_index("s")
    wid = cid * NS + tid

    def _zrow(i, carry):
        zb[i, :] = jnp.zeros((L,), jnp.float32)
        return carry

    lax.fori_loop(0, ZR1, _zrow, 0)

    def _zcopy(k, carry):
        pltpu.sync_copy(zb, den_sh.at[pl.ds(tid * RPT + k * ZR1, ZR1)])
        return carry

    lax.fori_loop(0, RPT // ZR1, _zcopy, 0)
    plsc.subcore_barrier()

    row0 = wid * NB1

    def _group(g, carry):
        grow = row0 + g * G1
        pltpu.sync_copy(src_hbm.at[pl.ds(grow, G1)], src1)
        pltpu.sync_copy(dst_hbm.at[pl.ds(grow, G1)], dst1)
        waits = []
        for j in range(G1):
            waits.append(pltpu.async_copy(
                ts_hbm.at[src1.at[j]], g1.at[pl.ds(j * B, B)], sem1))
            waits.append(pltpu.async_copy(
                td_hbm.at[dst1.at[j]], g2.at[pl.ds(j * B, B)], sem2))
        for d in waits:
            d.wait()

        def _edge(e, c2):
            v = g1[e, :] + g2[e, :]
            v = jnp.maximum(v, 0.2 * v)
            exb[e, :] = jnp.exp(v)
            return c2

        lax.fori_loop(0, G1 * B, _edge, 0, unroll=4)
        for j in range(G1):
            pltpu.sync_copy(exb.at[pl.ds(j * B, B)],
                            den_sh.at[dst1.at[j]], add=True)
        pltpu.sync_copy(exb, ex_hbm.at[pl.ds(grow * B, G1 * B)])
        return carry

    lax.fori_loop(0, NB1 // G1, _group, 0)
    plsc.subcore_barrier()
    pltpu.sync_copy(den_sh.at[pl.ds(tid * RPT, RPT)],
                    denp_hbm.at[cid, pl.ds(tid * RPT, RPT)])


@functools.partial(
    pl.kernel,
    out_type=jax.ShapeDtypeStruct((HEADS, NP, CHC), jnp.float32),
    mesh=_mesh,
    scratch_types=[
        pltpu.VMEM((4, B), jnp.int32),
        pltpu.VMEM((4, B), jnp.int32),
        pltpu.VMEM((B, L), jnp.float32),
        pltpu.VMEM((B, L), jnp.float32),
        pltpu.VMEM((B, L), jnp.float32),
        pltpu.VMEM((B, L), jnp.float32),
        pltpu.VMEM((B, CHC), jnp.float32),
        pltpu.VMEM((B, CHC), jnp.float32),
        pltpu.VMEM((B, CHC), jnp.float32),
        pltpu.VMEM((B, CHC), jnp.float32),
        pltpu.VMEM((ZR, CHC), jnp.float32),
        pltpu.SemaphoreType.DMA,
        pltpu.SemaphoreType.DMA,
        pltpu.SemaphoreType.DMA,
        pltpu.SemaphoreType.DMA,
        pltpu.SemaphoreType.DMA,
        pltpu.SemaphoreType.DMA,
        pltpu.VMEM_SHARED((NP, CHC), jnp.float32),
    ],
    compiler_params=pltpu.CompilerParams(
        use_tc_tiling_on_sc=False, needs_layout_passes=False),
)
def _sc_pass2(src_hbm, dst_hbm, ex_hbm, denr_hbm, hh0, hh1, hh2, hh3,
              out_hbm, src_i, dst_i, exb0, exb1, dg0, dg1, hg0, hg1,
              sb0, sb1, zb, semg0, semg1, sems0, sems1, semi0, semi1,
              acc_sh):
    cid = lax.axis_index("c")
    tid = lax.axis_index("s")
    exb = (exb0, exb1)
    dg = (dg0, dg1)
    hg = (hg0, hg1)
    sb = (sb0, sb1)
    semg = (semg0, semg1)
    sems = (sems0, sems1)
    semi = (semi0, semi1)

    def _zrow(i, carry):
        zb[i, pl.ds(0, L)] = jnp.zeros((L,), jnp.float32)
        zb[i, pl.ds(L, L)] = jnp.zeros((L,), jnp.float32)
        return carry

    lax.fori_loop(0, ZR, _zrow, 0)
    iota = lax.iota(jnp.int32, L)
    jfull = [jnp.full((L,), j, jnp.int32) for j in range(L)]

    def _sweep(hh_ref, slot):
        cslot = jnp.full((L,), slot, jnp.int32)

        def _zcopy(k, carry):
            pltpu.sync_copy(zb, acc_sh.at[pl.ds(tid * RPT + k * ZR, ZR)])
            return carry

        lax.fori_loop(0, RPT // ZR, _zcopy, 0)
        plsc.subcore_barrier()
        row0 = tid * NB2

        def _issue_g(g, p, k):
            pltpu.async_copy(
                ex_hbm.at[pl.ds((row0 + g) * B, B)], exb[p], semg[p])
            pltpu.async_copy(denr_hbm.at[dst_i.at[k]], dg[p], semg[p])
            pltpu.async_copy(hh_ref.at[src_i.at[k]], hg[p], semg[p])

        def _drain_g(p, k):
            pltpu.make_async_copy(
                ex_hbm.at[pl.ds(row0 * B, B)], exb[p], semg[p]).wait()
            pltpu.make_async_copy(
                denr_hbm.at[dst_i.at[k]], dg[p], semg[p]).wait()
            pltpu.make_async_copy(
                hh_ref.at[src_i.at[k]], hg[p], semg[p]).wait()

        def _drain_s(p):
            pltpu.make_async_copy(
                sb[p], acc_sh.at[dst_i.at[0]], sems[p]).wait()

        def _issue_i(g, k, p):
            pltpu.async_copy(src_hbm.at[row0 + g], src_i.at[k], semi[p])
            pltpu.async_copy(dst_hbm.at[row0 + g], dst_i.at[k], semi[p])

        def _drain_i(p, k):
            pltpu.make_async_copy(
                src_hbm.at[row0], src_i.at[k], semi[p]).wait()
            pltpu.make_async_copy(
                dst_hbm.at[row0], dst_i.at[k], semi[p]).wait()

        pltpu.sync_copy(src_hbm.at[row0], src_i.at[0])
        pltpu.sync_copy(dst_hbm.at[row0], dst_i.at[0])
        _issue_g(0, 0, 0)
        _issue_i(1, 1, 1)

        def _quad(q, carry):
            for k in range(4):
                g = 4 * q + k
                p = k % 2

                @pl.when(g >= 2)
                def _():
                    _drain_s(p)

                @pl.when(g + 2 < NB2)
                def _():
                    _issue_i(g + 2, (k + 2) % 4, p)

                _drain_g(p, k)

                def _grp(i):
                    ridx = i * L + iota
                    ev = plsc.load_gather(exb[p], [ridx, cslot])
                    dv = plsc.load_gather(dg[p], [ridx, cslot])
                    cv = ev * dv
                    for j in range(L):
                        e = i * L + j
                        cj = _lane_bcast(cv, jfull[j])
                        sb[p][e, pl.ds(0, L)] = hg[p][e, pl.ds(0, L)] * cj
                        sb[p][e, pl.ds(L, L)] = hg[p][e, pl.ds(L, L)] * cj

                plsc.parallel_loop(0, B // L, unroll=2)(_grp)
                pltpu.async_copy(sb[p], acc_sh.at[dst_i.at[k]],
                                 sems[p], add=True)

                @pl.when(g + 1 < NB2)
                def _():
                    _drain_i(1 - p, (k + 1) % 4)
                    _issue_g(g + 1, 1 - p, (k + 1) % 4)
            return carry

        lax.fori_loop(0, NB2 // 4, _quad, 0)
        _drain_s(0)
        _drain_s(1)
        plsc.subcore_barrier()
        pltpu.sync_copy(acc_sh.at[pl.ds(tid * RPT, RPT)],
                        out_hbm.at[slot, pl.ds(tid * RPT, RPT)])
        plsc.subcore_barrier()

    @pl.when(cid == 0)
    def _():
        _sweep(hh0, 0)
        _sweep(hh1, 1)

    @pl.when(cid == 1)
    def _():
        _sweep(hh2, 2)
        _sweep(hh3, 3)


def _attn_mat(a):
    m = jnp.zeros((HID, L), jnp.float32)
    for h in range(HEADS):
        m = m.at[h * CHC:(h + 1) * CHC, h].set(a[h])
    return m


def kernel(x, edge_index, W1e, b1e, W2e, b2e, Wc1, as1, ad1, bc1,
           Wc2, as2, ad2, bc2, Wd1, bd1, Wd2, bd2, Wr1, br1, Wr2, br2):
    x_pad = jnp.zeros((NP, 8), jnp.float32).at[:NN].set(x)
    loop_idx = jnp.arange(NN, dtype=jnp.int32)
    pad_idx = jnp.full((EP - EE - NN,), NN, jnp.int32)
    src = jnp.concatenate(
        [edge_index[0].astype(jnp.int32), loop_idx, pad_idx]).reshape(ER, B)
    dst = jnp.concatenate(
        [edge_index[1].astype(jnp.int32), loop_idx, pad_idx]).reshape(ER, B)

    h0, h1, h2, h3, ts, td = _enc(
        x_pad, W1e, b1e.reshape(1, HID), W2e, b2e.reshape(1, HID),
        Wc1, _attn_mat(as1), _attn_mat(ad1))
    ex1, denp1 = _sc_pass1(src, dst, ts, td)
    denr1 = _denmerge(denp1)
    out1 = _sc_pass2(src, dst, ex1, denr1, h0, h1, h2, h3)

    h0, h1, h2, h3, ts, td = _mid(
        out1, bc1.reshape(HEADS, CHC), Wc2, _attn_mat(as2), _attn_mat(ad2))
    ex2, denp2 = _sc_pass1(src, dst, ts, td)
    denr2 = _denmerge(denp2)
    out2 = _sc_pass2(src, dst, ex2, denr2, h0, h1, h2, h3)

    err, rep = _dec(
        out2, bc2.reshape(HEADS, CHC), Wd1, bd1.reshape(1, 64),
        Wd2, bd2.reshape(1, 4), Wr1, br1.reshape(1, 64),
        Wr2, br2.reshape(1, 1))
    return (err[:NN], rep[:NN])


# speedup vs baseline: 1.9593x; 1.1435x over previous
import functools

import jax
import jax.numpy as jnp
from jax import lax
from jax.experimental import pallas as pl
from jax.experimental.pallas import tpu as pltpu
from jax.experimental.pallas import tpu_sc as plsc

NN = 50000
EE = 800000
HEADS = 4
CHC = 32
HID = 128

L = 16
NC = 2
NS = 16
NW = NC * NS

NP = 50176
RB = 512
GRID = NP // RB
RPT = NP // NS

B = 128
EP = 851968
ER = EP // B
NB1 = EP // NW // B
NB2 = EP // NS // B
ZR = 98
ZR1 = 392
G1 = 4
G2 = 2

_mesh = plsc.VectorSubcoreMesh(
    core_axis_name="c", subcore_axis_name="s", num_cores=NC, num_subcores=NS)
def _lane_bcast(v, idx):
    return lax.gather(
        v, idx[:, None],
        dimension_numbers=lax.GatherDimensionNumbers(
            offset_dims=(), collapsed_slice_dims=(0,), start_index_map=(0,)),
        slice_sizes=(1,),
        mode=lax.GatherScatterMode.PROMISE_IN_BOUNDS)


def _enc_body(x_ref, w1_ref, b1_ref, w2_ref, b2_ref, wc_ref, as_ref, ad_ref,
              h0_ref, h1_ref, h2_ref, h3_ref, ts_ref, td_ref):
    h = jnp.dot(x_ref[...], w1_ref[...], preferred_element_type=jnp.float32)
    h = jnp.maximum(h + b1_ref[...], 0.0)
    h = jnp.dot(h, w2_ref[...], preferred_element_type=jnp.float32) + b2_ref[...]
    hh = jnp.dot(h, wc_ref[...], preferred_element_type=jnp.float32)
    ts_ref[...] = jnp.dot(hh, as_ref[...], preferred_element_type=jnp.float32)
    td_ref[...] = jnp.dot(hh, ad_ref[...], preferred_element_type=jnp.float32)
    h0_ref[...] = hh[:, 0 * CHC:1 * CHC]
    h1_ref[...] = hh[:, 1 * CHC:2 * CHC]
    h2_ref[...] = hh[:, 2 * CHC:3 * CHC]
    h3_ref[...] = hh[:, 3 * CHC:4 * CHC]


def _mid_body(o_ref, bc_ref, wc_ref, as_ref, ad_ref,
              h0_ref, h1_ref, h2_ref, h3_ref, ts_ref, td_ref):
    g = jnp.concatenate(
        [jnp.maximum(o_ref[h] + bc_ref[h], 0.0) for h in range(HEADS)], axis=1)
    hh = jnp.dot(g, wc_ref[...], preferred_element_type=jnp.float32)
    ts_ref[...] = jnp.dot(hh, as_ref[...], preferred_element_type=jnp.float32)
    td_ref[...] = jnp.dot(hh, ad_ref[...], preferred_element_type=jnp.float32)
    h0_ref[...] = hh[:, 0 * CHC:1 * CHC]
    h1_ref[...] = hh[:, 1 * CHC:2 * CHC]
    h2_ref[...] = hh[:, 2 * CHC:3 * CHC]
    h3_ref[...] = hh[:, 3 * CHC:4 * CHC]


def _dec_body(o_ref, bc_ref, wd1_ref, bd1_ref, wd2_ref, bd2_ref,
              wr1_ref, br1_ref, wr2_ref, br2_ref, err_ref, rep_ref):
    h2 = jnp.concatenate(
        [o_ref[h] + bc_ref[h] for h in range(HEADS)], axis=1)
    e = jnp.maximum(
        jnp.dot(h2, wd1_ref[...], preferred_element_type=jnp.float32)
        + bd1_ref[...], 0.0)
    err_ref[...] = jnp.dot(e, wd2_ref[...],
                           preferred_element_type=jnp.float32) + bd2_ref[...]
    r = jnp.maximum(
        jnp.dot(h2, wr1_ref[...], preferred_element_type=jnp.float32)
        + br1_ref[...], 0.0)
    rep_ref[...] = jnp.dot(r, wr2_ref[...],
                           preferred_element_type=jnp.float32) + br2_ref[...]


def _den_body(dp_ref, out_ref):
    out_ref[...] = 1.0 / (dp_ref[0] + dp_ref[1] + 1e-16)


def _full(shape):
    nd = len(shape)
    return pl.BlockSpec(shape, lambda i, _nd=nd: (0,) * _nd)


def _enc(x_pad, w1, b1, w2, b2, wc, a_s, a_d):
    return pl.pallas_call(
        _enc_body,
        grid=(GRID,),
        in_specs=[
            pl.BlockSpec((RB, 8), lambda i: (i, 0)),
            _full((8, HID)), _full((1, HID)), _full((HID, HID)),
            _full((1, HID)), _full((HID, HID)), _full((HID, L)),
            _full((HID, L)),
        ],
        out_specs=[pl.BlockSpec((RB, CHC), lambda i: (i, 0))] * HEADS
        + [pl.BlockSpec((RB, L), lambda i: (i, 0))] * 2,
        out_shape=[jax.ShapeDtypeStruct((NP, CHC), jnp.float32)] * HEADS
        + [jax.ShapeDtypeStruct((NP, L), jnp.float32)] * 2,
    )(x_pad, w1, b1, w2, b2, wc, a_s, a_d)


def _mid(o, bc, wc, a_s, a_d):
    return pl.pallas_call(
        _mid_body,
        grid=(GRID,),
        in_specs=[
            pl.BlockSpec((HEADS, RB, CHC), lambda i: (0, i, 0)),
            _full((HEADS, CHC)), _full((HID, HID)), _full((HID, L)),
            _full((HID, L)),
        ],
        out_specs=[pl.BlockSpec((RB, CHC), lambda i: (i, 0))] * HEADS
        + [pl.BlockSpec((RB, L), lambda i: (i, 0))] * 2,
        out_shape=[jax.ShapeDtypeStruct((NP, CHC), jnp.float32)] * HEADS
        + [jax.ShapeDtypeStruct((NP, L), jnp.float32)] * 2,
    )(o, bc, wc, a_s, a_d)


def _dec(o, bc, wd1, bd1, wd2, bd2, wr1, br1, wr2, br2):
    return pl.pallas_call(
        _dec_body,
        grid=(GRID,),
        in_specs=[
            pl.BlockSpec((HEADS, RB, CHC), lambda i: (0, i, 0)),
            _full((HEADS, CHC)), _full((HID, 64)), _full((1, 64)),
            _full((64, 4)), _full((1, 4)), _full((HID, 64)), _full((1, 64)),
            _full((64, 1)), _full((1, 1)),
        ],
        out_specs=[pl.BlockSpec((RB, 4), lambda i: (i, 0)),
                   pl.BlockSpec((RB, 1), lambda i: (i, 0))],
        out_shape=[jax.ShapeDtypeStruct((NP, 4), jnp.float32),
                   jax.ShapeDtypeStruct((NP, 1), jnp.float32)],
    )(o, bc, wd1, bd1, wd2, bd2, wr1, br1, wr2, br2)


def _denmerge(dp):
    return pl.pallas_call(
        _den_body,
        grid=(GRID,),
        in_specs=[pl.BlockSpec((NC, RB, L), lambda i: (0, i, 0))],
        out_specs=pl.BlockSpec((RB, L), lambda i: (i, 0)),
        out_shape=jax.ShapeDtypeStruct((NP, L), jnp.float32),
    )(dp)


@functools.partial(
    pl.kernel,
    out_type=(jax.ShapeDtypeStruct((EP, L), jnp.float32),
              jax.ShapeDtypeStruct((NC, NP, L), jnp.float32)),
    mesh=_mesh,
    scratch_types=[
        pltpu.VMEM((4, B), jnp.int32),
        pltpu.VMEM((4, B), jnp.int32),
        pltpu.VMEM((B, L), jnp.float32),
        pltpu.VMEM((B, L), jnp.float32),
        pltpu.VMEM((B, L), jnp.float32),
        pltpu.VMEM((B, L), jnp.float32),
        pltpu.VMEM((B, L), jnp.float32),
        pltpu.VMEM((B, L), jnp.float32),
        pltpu.VMEM((ZR1, L), jnp.float32),
        pltpu.SemaphoreType.DMA,
        pltpu.SemaphoreType.DMA,
        pltpu.SemaphoreType.DMA,
        pltpu.SemaphoreType.DMA,
        pltpu.SemaphoreType.DMA,
        pltpu.SemaphoreType.DMA,
        pltpu.SemaphoreType.DMA,
        pltpu.SemaphoreType.DMA,
        pltpu.VMEM_SHARED((NP, L), jnp.float32),
    ],
    compiler_params=pltpu.CompilerParams(
        use_tc_tiling_on_sc=False, needs_layout_passes=False),
)
def _sc_pass1(src_hbm, dst_hbm, ts_hbm, td_hbm, ex_hbm, denp_hbm,
              src_i, dst_i, g1a, g1b, g2a, g2b, exba, exbb, zb,
              semg0, semg1, sems0, sems1, semx0, semx1, semi0, semi1,
              den_sh):
    cid = lax.axis_index("c")
    tid = lax.axis_index("s")
    wid = cid * NS + tid
    g1 = (g1a, g1b)
    g2 = (g2a, g2b)
    exb = (exba, exbb)
    semg = (semg0, semg1)
    sems = (sems0, sems1)
    semx = (semx0, semx1)
    semi = (semi0, semi1)

    def _zrow(i, carry):
        zb[i, :] = jnp.zeros((L,), jnp.float32)
        return carry

    lax.fori_loop(0, ZR1, _zrow, 0)

    def _zcopy(k, carry):
        pltpu.sync_copy(zb, den_sh.at[pl.ds(tid * RPT + k * ZR1, ZR1)])
        return carry

    lax.fori_loop(0, RPT // ZR1, _zcopy, 0)
    plsc.subcore_barrier()

    row0 = wid * NB1

    def _issue_g(g, p, k):
        pltpu.async_copy(ts_hbm.at[src_i.at[k]], g1[p], semg[p])
        pltpu.async_copy(td_hbm.at[dst_i.at[k]], g2[p], semg[p])

    def _drain_g(p, k):
        pltpu.make_async_copy(
            ts_hbm.at[src_i.at[k]], g1[p], semg[p]).wait()
        pltpu.make_async_copy(
            td_hbm.at[dst_i.at[k]], g2[p], semg[p]).wait()

    def _drain_sx(p):
        pltpu.make_async_copy(
            exb[p], den_sh.at[dst_i.at[0]], sems[p]).wait()
        pltpu.make_async_copy(
            exb[p], ex_hbm.at[pl.ds(row0 * B, B)], semx[p]).wait()

    def _issue_i(g, k, p):
        pltpu.async_copy(src_hbm.at[row0 + g], src_i.at[k], semi[p])
        pltpu.async_copy(dst_hbm.at[row0 + g], dst_i.at[k], semi[p])

    def _drain_i(p, k):
        pltpu.make_async_copy(
            src_hbm.at[row0], src_i.at[k], semi[p]).wait()
        pltpu.make_async_copy(
            dst_hbm.at[row0], dst_i.at[k], semi[p]).wait()

    pltpu.sync_copy(src_hbm.at[row0], src_i.at[0])
    pltpu.sync_copy(dst_hbm.at[row0], dst_i.at[0])
    _issue_g(0, 0, 0)
    _issue_i(1, 1, 1)

    def _quad(q, carry):
        for k in range(4):
            g = 4 * q + k
            p = k % 2

            @pl.when(g >= 2)
            def _():
                _drain_sx(p)

            @pl.when(g + 2 < NB1)
            def _():
                _issue_i(g + 2, (k + 2) % 4, p)

            _drain_g(p, k)

            def _edge(e):
                v = g1[p][e, :] + g2[p][e, :]
                v = jnp.maximum(v, 0.2 * v)
                exb[p][e, :] = jnp.exp(v)

            plsc.parallel_loop(0, B, unroll=4)(_edge)
            pltpu.async_copy(exb[p], den_sh.at[dst_i.at[k]],
                             sems[p], add=True)
            pltpu.async_copy(
                exb[p], ex_hbm.at[pl.ds((row0 + g) * B, B)], semx[p])

            @pl.when(g + 1 < NB1)
            def _():
                _drain_i(1 - p, (k + 1) % 4)
                _issue_g(g + 1, 1 - p, (k + 1) % 4)
        return carry

    lax.fori_loop(0, NB1 // 4, _quad, 0)
    _drain_sx(0)
    _drain_sx(1)
    plsc.subcore_barrier()
    pltpu.sync_copy(den_sh.at[pl.ds(tid * RPT, RPT)],
                    denp_hbm.at[cid, pl.ds(tid * RPT, RPT)])


@functools.partial(
    pl.kernel,
    out_type=jax.ShapeDtypeStruct((HEADS, NP, CHC), jnp.float32),
    mesh=_mesh,
    scratch_types=[
        pltpu.VMEM((4, B), jnp.int32),
        pltpu.VMEM((4, B), jnp.int32),
        pltpu.VMEM((B, L), jnp.float32),
        pltpu.VMEM((B, L), jnp.float32),
        pltpu.VMEM((B, L), jnp.float32),
        pltpu.VMEM((B, L), jnp.float32),
        pltpu.VMEM((B, CHC), jnp.float32),
        pltpu.VMEM((B, CHC), jnp.float32),
        pltpu.VMEM((B, CHC), jnp.float32),
        pltpu.VMEM((B, CHC), jnp.float32),
        pltpu.VMEM((ZR, CHC), jnp.float32),
        pltpu.SemaphoreType.DMA,
        pltpu.SemaphoreType.DMA,
        pltpu.SemaphoreType.DMA,
        pltpu.SemaphoreType.DMA,
        pltpu.SemaphoreType.DMA,
        pltpu.SemaphoreType.DMA,
        pltpu.VMEM_SHARED((NP, CHC), jnp.float32),
    ],
    compiler_params=pltpu.CompilerParams(
        use_tc_tiling_on_sc=False, needs_layout_passes=False),
)
def _sc_pass2(src_hbm, dst_hbm, ex_hbm, denr_hbm, hh0, hh1, hh2, hh3,
              out_hbm, src_i, dst_i, exb0, exb1, dg0, dg1, hg0, hg1,
              sb0, sb1, zb, semg0, semg1, sems0, sems1, semi0, semi1,
              acc_sh):
    cid = lax.axis_index("c")
    tid = lax.axis_index("s")
    exb = (exb0, exb1)
    dg = (dg0, dg1)
    hg = (hg0, hg1)
    sb = (sb0, sb1)
    semg = (semg0, semg1)
    sems = (sems0, sems1)
    semi = (semi0, semi1)

    def _zrow(i, carry):
        zb[i, pl.ds(0, L)] = jnp.zeros((L,), jnp.float32)
        zb[i, pl.ds(L, L)] = jnp.zeros((L,), jnp.float32)
        return carry

    lax.fori_loop(0, ZR, _zrow, 0)
    iota = lax.iota(jnp.int32, L)
    jfull = [jnp.full((L,), j, jnp.int32) for j in range(L)]

    def _sweep(hh_ref, slot):
        cslot = jnp.full((L,), slot, jnp.int32)

        def _zcopy(k, carry):
            pltpu.sync_copy(zb, acc_sh.at[pl.ds(tid * RPT + k * ZR, ZR)])
            return carry

        lax.fori_loop(0, RPT // ZR, _zcopy, 0)
        plsc.subcore_barrier()
        row0 = tid * NB2

        def _issue_g(g, p, k):
            pltpu.async_copy(
                ex_hbm.at[pl.ds((row0 + g) * B, B)], exb[p], semg[p])
            pltpu.async_copy(denr_hbm.at[dst_i.at[k]], dg[p], semg[p])
            pltpu.async_copy(hh_ref.at[src_i.at[k]], hg[p], semg[p])

        def _drain_g(p, k):
            pltpu.make_async_copy(
                ex_hbm.at[pl.ds(row0 * B, B)], exb[p], semg[p]).wait()
            pltpu.make_async_copy(
                denr_hbm.at[dst_i.at[k]], dg[p], semg[p]).wait()
            pltpu.make_async_copy(
                hh_ref.at[src_i.at[k]], hg[p], semg[p]).wait()

        def _drain_s(p):
            pltpu.make_async_copy(
                sb[p], acc_sh.at[dst_i.at[0]], sems[p]).wait()

        def _issue_i(g, k, p):
            pltpu.async_copy(src_hbm.at[row0 + g], src_i.at[k], semi[p])
            pltpu.async_copy(dst_hbm.at[row0 + g], dst_i.at[k], semi[p])

        def _drain_i(p, k):
            pltpu.make_async_copy(
                src_hbm.at[row0], src_i.at[k], semi[p]).wait()
            pltpu.make_async_copy(
                dst_hbm.at[row0], dst_i.at[k], semi[p]).wait()

        pltpu.sync_copy(src_hbm.at[row0], src_i.at[0])
        pltpu.sync_copy(dst_hbm.at[row0], dst_i.at[0])
        _issue_g(0, 0, 0)
        _issue_i(1, 1, 1)

        def _quad(q, carry):
            for k in range(4):
                g = 4 * q + k
                p = k % 2

                @pl.when(g >= 2)
                def _():
                    _drain_s(p)

                @pl.when(g + 2 < NB2)
                def _():
                    _issue_i(g + 2, (k + 2) % 4, p)

                _drain_g(p, k)

                def _grp(i):
                    ridx = i * L + iota
                    ev = plsc.load_gather(exb[p], [ridx, cslot])
                    dv = plsc.load_gather(dg[p], [ridx, cslot])
                    cv = ev * dv
                    for j in range(L):
                        e = i * L + j
                        cj = _lane_bcast(cv, jfull[j])
                        sb[p][e, pl.ds(0, L)] = hg[p][e, pl.ds(0, L)] * cj
                        sb[p][e, pl.ds(L, L)] = hg[p][e, pl.ds(L, L)] * cj

                plsc.parallel_loop(0, B // L, unroll=2)(_grp)
                pltpu.async_copy(sb[p], acc_sh.at[dst_i.at[k]],
                                 sems[p], add=True)

                @pl.when(g + 1 < NB2)
                def _():
                    _drain_i(1 - p, (k + 1) % 4)
                    _issue_g(g + 1, 1 - p, (k + 1) % 4)
            return carry

        lax.fori_loop(0, NB2 // 4, _quad, 0)
        _drain_s(0)
        _drain_s(1)
        plsc.subcore_barrier()
        pltpu.sync_copy(acc_sh.at[pl.ds(tid * RPT, RPT)],
                        out_hbm.at[slot, pl.ds(tid * RPT, RPT)])
        plsc.subcore_barrier()

    @pl.when(cid == 0)
    def _():
        _sweep(hh0, 0)
        _sweep(hh1, 1)

    @pl.when(cid == 1)
    def _():
        _sweep(hh2, 2)
        _sweep(hh3, 3)


def _attn_mat(a):
    m = jnp.zeros((HID, L), jnp.float32)
    for h in range(HEADS):
        m = m.at[h * CHC:(h + 1) * CHC, h].set(a[h])
    return m


def kernel(x, edge_index, W1e, b1e, W2e, b2e, Wc1, as1, ad1, bc1,
           Wc2, as2, ad2, bc2, Wd1, bd1, Wd2, bd2, Wr1, br1, Wr2, br2):
    x_pad = jnp.zeros((NP, 8), jnp.float32).at[:NN].set(x)
    loop_idx = jnp.arange(NN, dtype=jnp.int32)
    pad_idx = jnp.full((EP - EE - NN,), NN, jnp.int32)
    src = jnp.concatenate(
        [edge_index[0].astype(jnp.int32), loop_idx, pad_idx]).reshape(ER, B)
    dst = jnp.concatenate(
        [edge_index[1].astype(jnp.int32), loop_idx, pad_idx]).reshape(ER, B)

    h0, h1, h2, h3, ts, td = _enc(
        x_pad, W1e, b1e.reshape(1, HID), W2e, b2e.reshape(1, HID),
        Wc1, _attn_mat(as1), _attn_mat(ad1))
    ex1, denp1 = _sc_pass1(src, dst, ts, td)
    denr1 = _denmerge(denp1)
    out1 = _sc_pass2(src, dst, ex1, denr1, h0, h1, h2, h3)

    h0, h1, h2, h3, ts, td = _mid(
        out1, bc1.reshape(HEADS, CHC), Wc2, _attn_mat(as2), _attn_mat(ad2))
    ex2, denp2 = _sc_pass1(src, dst, ts, td)
    denr2 = _denmerge(denp2)
    out2 = _sc_pass2(src, dst, ex2, denr2, h0, h1, h2, h3)

    err, rep = _dec(
        out2, bc2.reshape(HEADS, CHC), Wd1, bd1.reshape(1, 64),
        Wd2, bd2.reshape(1, 4), Wr1, br1.reshape(1, 64),
        Wr2, br2.reshape(1, 1))
    return (err[:NN], rep[:NN])


# speedup vs baseline: 2.1692x; 1.1071x over previous
import functools

import jax
import jax.numpy as jnp
from jax import lax
from jax.experimental import pallas as pl
from jax.experimental.pallas import tpu as pltpu
from jax.experimental.pallas import tpu_sc as plsc

NN = 50000
EE = 800000
HEADS = 4
CHC = 32
HID = 128

L = 16
NC = 2
NS = 16
NW = NC * NS

NP = 50176
RB = 512
GRID = NP // RB
RPT = NP // NS

B = 128
EP = 851968
ER = EP // B
NB1 = EP // NW // B
NB2 = EP // NS // B
ZR = 98
ZR1 = 392
G1 = 4
G2 = 2

_mesh = plsc.VectorSubcoreMesh(
    core_axis_name="c", subcore_axis_name="s", num_cores=NC, num_subcores=NS)
def _lane_bcast(v, idx):
    return lax.gather(
        v, idx[:, None],
        dimension_numbers=lax.GatherDimensionNumbers(
            offset_dims=(), collapsed_slice_dims=(0,), start_index_map=(0,)),
        slice_sizes=(1,),
        mode=lax.GatherScatterMode.PROMISE_IN_BOUNDS)


def _enc_body(x_ref, w1_ref, b1_ref, w2_ref, b2_ref, wc_ref, as_ref, ad_ref,
              h0_ref, h1_ref, h2_ref, h3_ref, ts_ref, td_ref):
    h = jnp.dot(x_ref[...], w1_ref[...], preferred_element_type=jnp.float32)
    h = jnp.maximum(h + b1_ref[...], 0.0)
    h = jnp.dot(h, w2_ref[...], preferred_element_type=jnp.float32) + b2_ref[...]
    hh = jnp.dot(h, wc_ref[...], preferred_element_type=jnp.float32)
    ts_ref[...] = jnp.dot(hh, as_ref[...], preferred_element_type=jnp.float32)
    td_ref[...] = jnp.dot(hh, ad_ref[...], preferred_element_type=jnp.float32)
    h0_ref[...] = hh[:, 0 * CHC:1 * CHC]
    h1_ref[...] = hh[:, 1 * CHC:2 * CHC]
    h2_ref[...] = hh[:, 2 * CHC:3 * CHC]
    h3_ref[...] = hh[:, 3 * CHC:4 * CHC]


def _mid_body(o_ref, dn_ref, bc_ref, wc_ref, as_ref, ad_ref,
              h0_ref, h1_ref, h2_ref, h3_ref, ts_ref, td_ref):
    g = jnp.concatenate(
        [jnp.maximum(o_ref[h] * dn_ref[:, h:h + 1] + bc_ref[h], 0.0)
         for h in range(HEADS)], axis=1)
    hh = jnp.dot(g, wc_ref[...], preferred_element_type=jnp.float32)
    ts_ref[...] = jnp.dot(hh, as_ref[...], preferred_element_type=jnp.float32)
    td_ref[...] = jnp.dot(hh, ad_ref[...], preferred_element_type=jnp.float32)
    h0_ref[...] = hh[:, 0 * CHC:1 * CHC]
    h1_ref[...] = hh[:, 1 * CHC:2 * CHC]
    h2_ref[...] = hh[:, 2 * CHC:3 * CHC]
    h3_ref[...] = hh[:, 3 * CHC:4 * CHC]


def _dec_body(o_ref, dn_ref, bc_ref, wd1_ref, bd1_ref, wd2_ref, bd2_ref,
              wr1_ref, br1_ref, wr2_ref, br2_ref, err_ref, rep_ref):
    h2 = jnp.concatenate(
        [o_ref[h] * dn_ref[:, h:h + 1] + bc_ref[h] for h in range(HEADS)],
        axis=1)
    e = jnp.maximum(
        jnp.dot(h2, wd1_ref[...], preferred_element_type=jnp.float32)
        + bd1_ref[...], 0.0)
    err_ref[...] = jnp.dot(e, wd2_ref[...],
                           preferred_element_type=jnp.float32) + bd2_ref[...]
    r = jnp.maximum(
        jnp.dot(h2, wr1_ref[...], preferred_element_type=jnp.float32)
        + br1_ref[...], 0.0)
    rep_ref[...] = jnp.dot(r, wr2_ref[...],
                           preferred_element_type=jnp.float32) + br2_ref[...]


def _den_body(dp_ref, out_ref):
    out_ref[...] = 1.0 / (dp_ref[0] + dp_ref[1] + 1e-16)


def _full(shape):
    nd = len(shape)
    return pl.BlockSpec(shape, lambda i, _nd=nd: (0,) * _nd)


def _enc(x_pad, w1, b1, w2, b2, wc, a_s, a_d):
    return pl.pallas_call(
        _enc_body,
        grid=(GRID,),
        in_specs=[
            pl.BlockSpec((RB, 8), lambda i: (i, 0)),
            _full((8, HID)), _full((1, HID)), _full((HID, HID)),
            _full((1, HID)), _full((HID, HID)), _full((HID, L)),
            _full((HID, L)),
        ],
        out_specs=[pl.BlockSpec((RB, CHC), lambda i: (i, 0))] * HEADS
        + [pl.BlockSpec((RB, L), lambda i: (i, 0))] * 2,
        out_shape=[jax.ShapeDtypeStruct((NP, CHC), jnp.float32)] * HEADS
        + [jax.ShapeDtypeStruct((NP, L), jnp.float32)] * 2,
    )(x_pad, w1, b1, w2, b2, wc, a_s, a_d)


def _mid(o, dn, bc, wc, a_s, a_d):
    return pl.pallas_call(
        _mid_body,
        grid=(GRID,),
        in_specs=[
            pl.BlockSpec((HEADS, RB, CHC), lambda i: (0, i, 0)),
            pl.BlockSpec((RB, L), lambda i: (i, 0)),
            _full((HEADS, CHC)), _full((HID, HID)), _full((HID, L)),
            _full((HID, L)),
        ],
        out_specs=[pl.BlockSpec((RB, CHC), lambda i: (i, 0))] * HEADS
        + [pl.BlockSpec((RB, L), lambda i: (i, 0))] * 2,
        out_shape=[jax.ShapeDtypeStruct((NP, CHC), jnp.float32)] * HEADS
        + [jax.ShapeDtypeStruct((NP, L), jnp.float32)] * 2,
    )(o, dn, bc, wc, a_s, a_d)


def _dec(o, dn, bc, wd1, bd1, wd2, bd2, wr1, br1, wr2, br2):
    return pl.pallas_call(
        _dec_body,
        grid=(GRID,),
        in_specs=[
            pl.BlockSpec((HEADS, RB, CHC), lambda i: (0, i, 0)),
            pl.BlockSpec((RB, L), lambda i: (i, 0)),
            _full((HEADS, CHC)), _full((HID, 64)), _full((1, 64)),
            _full((64, 4)), _full((1, 4)), _full((HID, 64)), _full((1, 64)),
            _full((64, 1)), _full((1, 1)),
        ],
        out_specs=[pl.BlockSpec((RB, 4), lambda i: (i, 0)),
                   pl.BlockSpec((RB, 1), lambda i: (i, 0))],
        out_shape=[jax.ShapeDtypeStruct((NP, 4), jnp.float32),
                   jax.ShapeDtypeStruct((NP, 1), jnp.float32)],
    )(o, dn, bc, wd1, bd1, wd2, bd2, wr1, br1, wr2, br2)


def _denmerge(dp):
    return pl.pallas_call(
        _den_body,
        grid=(GRID,),
        in_specs=[pl.BlockSpec((NC, RB, L), lambda i: (0, i, 0))],
        out_specs=pl.BlockSpec((RB, L), lambda i: (i, 0)),
        out_shape=jax.ShapeDtypeStruct((NP, L), jnp.float32),
    )(dp)


@functools.partial(
    pl.kernel,
    out_type=(jax.ShapeDtypeStruct((EP, L), jnp.float32),
              jax.ShapeDtypeStruct((NC, NP, L), jnp.float32)),
    mesh=_mesh,
    scratch_types=[
        pltpu.VMEM((4, B), jnp.int32),
        pltpu.VMEM((4, B), jnp.int32),
        pltpu.VMEM((B, L), jnp.float32),
        pltpu.VMEM((B, L), jnp.float32),
        pltpu.VMEM((B, L), jnp.float32),
        pltpu.VMEM((B, L), jnp.float32),
        pltpu.VMEM((B, L), jnp.float32),
        pltpu.VMEM((B, L), jnp.float32),
        pltpu.VMEM((ZR1, L), jnp.float32),
        pltpu.SemaphoreType.DMA,
        pltpu.SemaphoreType.DMA,
        pltpu.SemaphoreType.DMA,
        pltpu.SemaphoreType.DMA,
        pltpu.SemaphoreType.DMA,
        pltpu.SemaphoreType.DMA,
        pltpu.SemaphoreType.DMA,
        pltpu.SemaphoreType.DMA,
        pltpu.VMEM_SHARED((NP, L), jnp.float32),
    ],
    compiler_params=pltpu.CompilerParams(
        use_tc_tiling_on_sc=False, needs_layout_passes=False),
)
def _sc_pass1(src_hbm, dst_hbm, ts_hbm, td_hbm, ex_hbm, denp_hbm,
              src_i, dst_i, g1a, g1b, g2a, g2b, exba, exbb, zb,
              semg0, semg1, sems0, sems1, semx0, semx1, semi0, semi1,
              den_sh):
    cid = lax.axis_index("c")
    tid = lax.axis_index("s")
    wid = cid * NS + tid
    g1 = (g1a, g1b)
    g2 = (g2a, g2b)
    exb = (exba, exbb)
    semg = (semg0, semg1)
    sems = (sems0, sems1)
    semx = (semx0, semx1)
    semi = (semi0, semi1)

    def _zrow(i, carry):
        zb[i, :] = jnp.zeros((L,), jnp.float32)
        return carry

    lax.fori_loop(0, ZR1, _zrow, 0)

    def _zcopy(k, carry):
        pltpu.sync_copy(zb, den_sh.at[pl.ds(tid * RPT + k * ZR1, ZR1)])
        return carry

    lax.fori_loop(0, RPT // ZR1, _zcopy, 0)
    plsc.subcore_barrier()

    row0 = wid * NB1

    def _issue_g(g, p, k):
        pltpu.async_copy(ts_hbm.at[src_i.at[k]], g1[p], semg[p])
        pltpu.async_copy(td_hbm.at[dst_i.at[k]], g2[p], semg[p])

    def _drain_g(p, k):
        pltpu.make_async_copy(
            ts_hbm.at[src_i.at[k]], g1[p], semg[p]).wait()
        pltpu.make_async_copy(
            td_hbm.at[dst_i.at[k]], g2[p], semg[p]).wait()

    def _drain_sx(p):
        pltpu.make_async_copy(
            exb[p], den_sh.at[dst_i.at[0]], sems[p]).wait()
        pltpu.make_async_copy(
            exb[p], ex_hbm.at[pl.ds(row0 * B, B)], semx[p]).wait()

    def _issue_i(g, k, p):
        pltpu.async_copy(src_hbm.at[row0 + g], src_i.at[k], semi[p])
        pltpu.async_copy(dst_hbm.at[row0 + g], dst_i.at[k], semi[p])

    def _drain_i(p, k):
        pltpu.make_async_copy(
            src_hbm.at[row0], src_i.at[k], semi[p]).wait()
        pltpu.make_async_copy(
            dst_hbm.at[row0], dst_i.at[k], semi[p]).wait()

    pltpu.sync_copy(src_hbm.at[row0], src_i.at[0])
    pltpu.sync_copy(dst_hbm.at[row0], dst_i.at[0])
    _issue_g(0, 0, 0)
    _issue_i(1, 1, 1)

    def _quad(q, carry):
        for k in range(4):
            g = 4 * q + k
            p = k % 2

            @pl.when(g >= 2)
            def _():
                _drain_sx(p)

            @pl.when(g + 2 < NB1)
            def _():
                _issue_i(g + 2, (k + 2) % 4, p)

            _drain_g(p, k)

            def _edge(e):
                v = g1[p][e, :] + g2[p][e, :]
                v = jnp.maximum(v, 0.2 * v)
                exb[p][e, :] = jnp.exp(v)

            plsc.parallel_loop(0, B, unroll=4)(_edge)
            pltpu.async_copy(exb[p], den_sh.at[dst_i.at[k]],
                             sems[p], add=True)
            pltpu.async_copy(
                exb[p], ex_hbm.at[pl.ds((row0 + g) * B, B)], semx[p])

            @pl.when(g + 1 < NB1)
            def _():
                _drain_i(1 - p, (k + 1) % 4)
                _issue_g(g + 1, 1 - p, (k + 1) % 4)
        return carry

    lax.fori_loop(0, NB1 // 4, _quad, 0)
    _drain_sx(0)
    _drain_sx(1)
    plsc.subcore_barrier()
    pltpu.sync_copy(den_sh.at[pl.ds(tid * RPT, RPT)],
                    denp_hbm.at[cid, pl.ds(tid * RPT, RPT)])


@functools.partial(
    pl.kernel,
    out_type=jax.ShapeDtypeStruct((HEADS, NP, CHC), jnp.float32),
    mesh=_mesh,
    scratch_types=[
        pltpu.VMEM((4, B), jnp.int32),
        pltpu.VMEM((4, B), jnp.int32),
        pltpu.VMEM((B, L), jnp.float32),
        pltpu.VMEM((B, L), jnp.float32),
        pltpu.VMEM((B, CHC), jnp.float32),
        pltpu.VMEM((B, CHC), jnp.float32),
        pltpu.VMEM((B, CHC), jnp.float32),
        pltpu.VMEM((B, CHC), jnp.float32),
        pltpu.VMEM((ZR, CHC), jnp.float32),
        pltpu.SemaphoreType.DMA,
        pltpu.SemaphoreType.DMA,
        pltpu.SemaphoreType.DMA,
        pltpu.SemaphoreType.DMA,
        pltpu.SemaphoreType.DMA,
        pltpu.SemaphoreType.DMA,
        pltpu.VMEM_SHARED((NP, CHC), jnp.float32),
    ],
    compiler_params=pltpu.CompilerParams(
        use_tc_tiling_on_sc=False, needs_layout_passes=False),
)
def _sc_pass2(src_hbm, dst_hbm, ex_hbm, hh0, hh1, hh2, hh3,
              out_hbm, src_i, dst_i, exb0, exb1, hg0, hg1,
              sb0, sb1, zb, semg0, semg1, sems0, sems1, semi0, semi1,
              acc_sh):
    cid = lax.axis_index("c")
    tid = lax.axis_index("s")
    exb = (exb0, exb1)
    hg = (hg0, hg1)
    sb = (sb0, sb1)
    semg = (semg0, semg1)
    sems = (sems0, sems1)
    semi = (semi0, semi1)

    def _zrow(i, carry):
        zb[i, pl.ds(0, L)] = jnp.zeros((L,), jnp.float32)
        zb[i, pl.ds(L, L)] = jnp.zeros((L,), jnp.float32)
        return carry

    lax.fori_loop(0, ZR, _zrow, 0)
    iota = lax.iota(jnp.int32, L)
    jfull = [jnp.full((L,), j, jnp.int32) for j in range(L)]

    def _sweep(hh_ref, slot):
        cslot = jnp.full((L,), slot, jnp.int32)

        def _zcopy(k, carry):
            pltpu.sync_copy(zb, acc_sh.at[pl.ds(tid * RPT + k * ZR, ZR)])
            return carry

        lax.fori_loop(0, RPT // ZR, _zcopy, 0)
        plsc.subcore_barrier()
        row0 = tid * NB2

        def _issue_g(g, p, k):
            pltpu.async_copy(
                ex_hbm.at[pl.ds((row0 + g) * B, B)], exb[p], semg[p])
            pltpu.async_copy(hh_ref.at[src_i.at[k]], hg[p], semg[p])

        def _drain_g(p, k):
            pltpu.make_async_copy(
                ex_hbm.at[pl.ds(row0 * B, B)], exb[p], semg[p]).wait()
            pltpu.make_async_copy(
                hh_ref.at[src_i.at[k]], hg[p], semg[p]).wait()

        def _drain_s(p):
            pltpu.make_async_copy(
                sb[p], acc_sh.at[dst_i.at[0]], sems[p]).wait()

        def _issue_i(g, k, p):
            pltpu.async_copy(src_hbm.at[row0 + g], src_i.at[k], semi[p])
            pltpu.async_copy(dst_hbm.at[row0 + g], dst_i.at[k], semi[p])

        def _drain_i(p, k):
            pltpu.make_async_copy(
                src_hbm.at[row0], src_i.at[k], semi[p]).wait()
            pltpu.make_async_copy(
                dst_hbm.at[row0], dst_i.at[k], semi[p]).wait()

        pltpu.sync_copy(src_hbm.at[row0], src_i.at[0])
        pltpu.sync_copy(dst_hbm.at[row0], dst_i.at[0])
        _issue_g(0, 0, 0)
        _issue_i(1, 1, 1)

        def _quad(q, carry):
            for k in range(4):
                g = 4 * q + k
                p = k % 2

                @pl.when(g >= 2)
                def _():
                    _drain_s(p)

                @pl.when(g + 2 < NB2)
                def _():
                    _issue_i(g + 2, (k + 2) % 4, p)

                _drain_g(p, k)

                def _grp(i):
                    ridx = i * L + iota
                    cv = plsc.load_gather(exb[p], [ridx, cslot])
                    for j in range(L):
                        e = i * L + j
                        cj = _lane_bcast(cv, jfull[j])
                        sb[p][e, pl.ds(0, L)] = hg[p][e, pl.ds(0, L)] * cj
                        sb[p][e, pl.ds(L, L)] = hg[p][e, pl.ds(L, L)] * cj

                plsc.parallel_loop(0, B // L, unroll=2)(_grp)
                pltpu.async_copy(sb[p], acc_sh.at[dst_i.at[k]],
                                 sems[p], add=True)

                @pl.when(g + 1 < NB2)
                def _():
                    _drain_i(1 - p, (k + 1) % 4)
                    _issue_g(g + 1, 1 - p, (k + 1) % 4)
            return carry

        lax.fori_loop(0, NB2 // 4, _quad, 0)
        _drain_s(0)
        _drain_s(1)
        plsc.subcore_barrier()
        pltpu.sync_copy(acc_sh.at[pl.ds(tid * RPT, RPT)],
                        out_hbm.at[slot, pl.ds(tid * RPT, RPT)])
        plsc.subcore_barrier()

    @pl.when(cid == 0)
    def _():
        _sweep(hh0, 0)
        _sweep(hh1, 1)

    @pl.when(cid == 1)
    def _():
        _sweep(hh2, 2)
        _sweep(hh3, 3)


def _attn_mat(a):
    m = jnp.zeros((HID, L), jnp.float32)
    for h in range(HEADS):
        m = m.at[h * CHC:(h + 1) * CHC, h].set(a[h])
    return m


def kernel(x, edge_index, W1e, b1e, W2e, b2e, Wc1, as1, ad1, bc1,
           Wc2, as2, ad2, bc2, Wd1, bd1, Wd2, bd2, Wr1, br1, Wr2, br2):
    x_pad = jnp.zeros((NP, 8), jnp.float32).at[:NN].set(x)
    loop_idx = jnp.arange(NN, dtype=jnp.int32)
    pad_idx = jnp.full((EP - EE - NN,), NN, jnp.int32)
    src = jnp.concatenate(
        [edge_index[0].astype(jnp.int32), loop_idx, pad_idx]).reshape(ER, B)
    dst = jnp.concatenate(
        [edge_index[1].astype(jnp.int32), loop_idx, pad_idx]).reshape(ER, B)

    h0, h1, h2, h3, ts, td = _enc(
        x_pad, W1e, b1e.reshape(1, HID), W2e, b2e.reshape(1, HID),
        Wc1, _attn_mat(as1), _attn_mat(ad1))
    ex1, denp1 = _sc_pass1(src, dst, ts, td)
    denr1 = _denmerge(denp1)
    out1 = _sc_pass2(src, dst, ex1, h0, h1, h2, h3)

    h0, h1, h2, h3, ts, td = _mid(
        out1, denr1, bc1.reshape(HEADS, CHC), Wc2,
        _attn_mat(as2), _attn_mat(ad2))
    ex2, denp2 = _sc_pass1(src, dst, ts, td)
    denr2 = _denmerge(denp2)
    out2 = _sc_pass2(src, dst, ex2, h0, h1, h2, h3)

    err, rep = _dec(
        out2, denr2, bc2.reshape(HEADS, CHC), Wd1, bd1.reshape(1, 64),
        Wd2, bd2.reshape(1, 4), Wr1, br1.reshape(1, 64),
        Wr2, br2.reshape(1, 1))
    return (err[:NN], rep[:NN])


# speedup vs baseline: 2.1737x; 1.0021x over previous
import functools

import jax
import jax.numpy as jnp
from jax import lax
from jax.experimental import pallas as pl
from jax.experimental.pallas import tpu as pltpu
from jax.experimental.pallas import tpu_sc as plsc

NN = 50000
EE = 800000
HEADS = 4
CHC = 32
HID = 128

L = 16
NC = 2
NS = 16
NW = NC * NS

NP = 50176
RB = 512
GRID = NP // RB
RPT = NP // NS

B = 128
EP = 851968
ER = EP // B
NB1 = EP // NW // B
NB2 = EP // NS // B
ZR = 98
ZR1 = 392
G1 = 4
G2 = 2

_mesh = plsc.VectorSubcoreMesh(
    core_axis_name="c", subcore_axis_name="s", num_cores=NC, num_subcores=NS)
def _lane_bcast(v, idx):
    return lax.gather(
        v, idx[:, None],
        dimension_numbers=lax.GatherDimensionNumbers(
            offset_dims=(), collapsed_slice_dims=(0,), start_index_map=(0,)),
        slice_sizes=(1,),
        mode=lax.GatherScatterMode.PROMISE_IN_BOUNDS)


def _enc_body(x_ref, w1_ref, b1_ref, w2_ref, b2_ref, wc_ref, as_ref, ad_ref,
              h0_ref, h1_ref, h2_ref, h3_ref, ts_ref, td_ref):
    h = jnp.dot(x_ref[...], w1_ref[...], preferred_element_type=jnp.float32)
    h = jnp.maximum(h + b1_ref[...], 0.0)
    h = jnp.dot(h, w2_ref[...], preferred_element_type=jnp.float32) + b2_ref[...]
    hh = jnp.dot(h, wc_ref[...], preferred_element_type=jnp.float32)
    ts_ref[...] = jnp.dot(hh, as_ref[...], preferred_element_type=jnp.float32)
    td_ref[...] = jnp.dot(hh, ad_ref[...], preferred_element_type=jnp.float32)
    h0_ref[...] = hh[:, 0 * CHC:1 * CHC]
    h1_ref[...] = hh[:, 1 * CHC:2 * CHC]
    h2_ref[...] = hh[:, 2 * CHC:3 * CHC]
    h3_ref[...] = hh[:, 3 * CHC:4 * CHC]


def _mid_body(o_ref, dp_ref, bc_ref, wc_ref, as_ref, ad_ref,
              h0_ref, h1_ref, h2_ref, h3_ref, ts_ref, td_ref):
    dn = 1.0 / (dp_ref[0] + dp_ref[1] + 1e-16)
    g = jnp.concatenate(
        [jnp.maximum(o_ref[h] * dn[:, h:h + 1] + bc_ref[h], 0.0)
         for h in range(HEADS)], axis=1)
    hh = jnp.dot(g, wc_ref[...], preferred_element_type=jnp.float32)
    ts_ref[...] = jnp.dot(hh, as_ref[...], preferred_element_type=jnp.float32)
    td_ref[...] = jnp.dot(hh, ad_ref[...], preferred_element_type=jnp.float32)
    h0_ref[...] = hh[:, 0 * CHC:1 * CHC]
    h1_ref[...] = hh[:, 1 * CHC:2 * CHC]
    h2_ref[...] = hh[:, 2 * CHC:3 * CHC]
    h3_ref[...] = hh[:, 3 * CHC:4 * CHC]


def _dec_body(o_ref, dp_ref, bc_ref, wd1_ref, bd1_ref, wd2_ref, bd2_ref,
              wr1_ref, br1_ref, wr2_ref, br2_ref, err_ref, rep_ref):
    dn = 1.0 / (dp_ref[0] + dp_ref[1] + 1e-16)
    h2 = jnp.concatenate(
        [o_ref[h] * dn[:, h:h + 1] + bc_ref[h] for h in range(HEADS)],
        axis=1)
    e = jnp.maximum(
        jnp.dot(h2, wd1_ref[...], preferred_element_type=jnp.float32)
        + bd1_ref[...], 0.0)
    err_ref[...] = jnp.dot(e, wd2_ref[...],
                           preferred_element_type=jnp.float32) + bd2_ref[...]
    r = jnp.maximum(
        jnp.dot(h2, wr1_ref[...], preferred_element_type=jnp.float32)
        + br1_ref[...], 0.0)
    rep_ref[...] = jnp.dot(r, wr2_ref[...],
                           preferred_element_type=jnp.float32) + br2_ref[...]


def _full(shape):
    nd = len(shape)
    return pl.BlockSpec(shape, lambda i, _nd=nd: (0,) * _nd)


def _enc(x_pad, w1, b1, w2, b2, wc, a_s, a_d):
    return pl.pallas_call(
        _enc_body,
        grid=(GRID,),
        in_specs=[
            pl.BlockSpec((RB, 8), lambda i: (i, 0)),
            _full((8, HID)), _full((1, HID)), _full((HID, HID)),
            _full((1, HID)), _full((HID, HID)), _full((HID, L)),
            _full((HID, L)),
        ],
        out_specs=[pl.BlockSpec((RB, CHC), lambda i: (i, 0))] * HEADS
        + [pl.BlockSpec((RB, L), lambda i: (i, 0))] * 2,
        out_shape=[jax.ShapeDtypeStruct((NP, CHC), jnp.float32)] * HEADS
        + [jax.ShapeDtypeStruct((NP, L), jnp.float32)] * 2,
    )(x_pad, w1, b1, w2, b2, wc, a_s, a_d)


def _mid(o, dn, bc, wc, a_s, a_d):
    return pl.pallas_call(
        _mid_body,
        grid=(GRID,),
        in_specs=[
            pl.BlockSpec((HEADS, RB, CHC), lambda i: (0, i, 0)),
            pl.BlockSpec((NC, RB, L), lambda i: (0, i, 0)),
            _full((HEADS, CHC)), _full((HID, HID)), _full((HID, L)),
            _full((HID, L)),
        ],
        out_specs=[pl.BlockSpec((RB, CHC), lambda i: (i, 0))] * HEADS
        + [pl.BlockSpec((RB, L), lambda i: (i, 0))] * 2,
        out_shape=[jax.ShapeDtypeStruct((NP, CHC), jnp.float32)] * HEADS
        + [jax.ShapeDtypeStruct((NP, L), jnp.float32)] * 2,
    )(o, dn, bc, wc, a_s, a_d)


def _dec(o, dn, bc, wd1, bd1, wd2, bd2, wr1, br1, wr2, br2):
    return pl.pallas_call(
        _dec_body,
        grid=(GRID,),
        in_specs=[
            pl.BlockSpec((HEADS, RB, CHC), lambda i: (0, i, 0)),
            pl.BlockSpec((NC, RB, L), lambda i: (0, i, 0)),
            _full((HEADS, CHC)), _full((HID, 64)), _full((1, 64)),
            _full((64, 4)), _full((1, 4)), _full((HID, 64)), _full((1, 64)),
            _full((64, 1)), _full((1, 1)),
        ],
        out_specs=[pl.BlockSpec((RB, 4), lambda i: (i, 0)),
                   pl.BlockSpec((RB, 1), lambda i: (i, 0))],
        out_shape=[jax.ShapeDtypeStruct((NP, 4), jnp.float32),
                   jax.ShapeDtypeStruct((NP, 1), jnp.float32)],
    )(o, dn, bc, wd1, bd1, wd2, bd2, wr1, br1, wr2, br2)


@functools.partial(
    pl.kernel,
    out_type=(jax.ShapeDtypeStruct((EP, L), jnp.float32),
              jax.ShapeDtypeStruct((NC, NP, L), jnp.float32)),
    mesh=_mesh,
    scratch_types=[
        pltpu.VMEM((4, B), jnp.int32),
        pltpu.VMEM((4, B), jnp.int32),
        pltpu.VMEM((B, L), jnp.float32),
        pltpu.VMEM((B, L), jnp.float32),
        pltpu.VMEM((B, L), jnp.float32),
        pltpu.VMEM((B, L), jnp.float32),
        pltpu.VMEM((B, L), jnp.float32),
        pltpu.VMEM((B, L), jnp.float32),
        pltpu.VMEM((ZR1, L), jnp.float32),
        pltpu.SemaphoreType.DMA,
        pltpu.SemaphoreType.DMA,
        pltpu.SemaphoreType.DMA,
        pltpu.SemaphoreType.DMA,
        pltpu.SemaphoreType.DMA,
        pltpu.SemaphoreType.DMA,
        pltpu.SemaphoreType.DMA,
        pltpu.SemaphoreType.DMA,
        pltpu.VMEM_SHARED((NP, L), jnp.float32),
    ],
    compiler_params=pltpu.CompilerParams(
        use_tc_tiling_on_sc=False, needs_layout_passes=False),
)
def _sc_pass1(src_hbm, dst_hbm, ts_hbm, td_hbm, ex_hbm, denp_hbm,
              src_i, dst_i, g1a, g1b, g2a, g2b, exba, exbb, zb,
              semg0, semg1, sems0, sems1, semx0, semx1, semi0, semi1,
              den_sh):
    cid = lax.axis_index("c")
    tid = lax.axis_index("s")
    wid = cid * NS + tid
    g1 = (g1a, g1b)
    g2 = (g2a, g2b)
    exb = (exba, exbb)
    semg = (semg0, semg1)
    sems = (sems0, sems1)
    semx = (semx0, semx1)
    semi = (semi0, semi1)

    def _zrow(i, carry):
        zb[i, :] = jnp.zeros((L,), jnp.float32)
        return carry

    lax.fori_loop(0, ZR1, _zrow, 0)

    def _zcopy(k, carry):
        pltpu.sync_copy(zb, den_sh.at[pl.ds(tid * RPT + k * ZR1, ZR1)])
        return carry

    lax.fori_loop(0, RPT // ZR1, _zcopy, 0)
    plsc.subcore_barrier()

    row0 = wid * NB1

    def _issue_g(g, p, k):
        pltpu.async_copy(ts_hbm.at[src_i.at[k]], g1[p], semg[p])
        pltpu.async_copy(td_hbm.at[dst_i.at[k]], g2[p], semg[p])

    def _drain_g(p, k):
        pltpu.make_async_copy(
            ts_hbm.at[src_i.at[k]], g1[p], semg[p]).wait()
        pltpu.make_async_copy(
            td_hbm.at[dst_i.at[k]], g2[p], semg[p]).wait()

    def _drain_sx(p):
        pltpu.make_async_copy(
            exb[p], den_sh.at[dst_i.at[0]], sems[p]).wait()
        pltpu.make_async_copy(
            exb[p], ex_hbm.at[pl.ds(row0 * B, B)], semx[p]).wait()

    def _issue_i(g, k, p):
        pltpu.async_copy(src_hbm.at[row0 + g], src_i.at[k], semi[p])
        pltpu.async_copy(dst_hbm.at[row0 + g], dst_i.at[k], semi[p])

    def _drain_i(p, k):
        pltpu.make_async_copy(
            src_hbm.at[row0], src_i.at[k], semi[p]).wait()
        pltpu.make_async_copy(
            dst_hbm.at[row0], dst_i.at[k], semi[p]).wait()

    pltpu.sync_copy(src_hbm.at[row0], src_i.at[0])
    pltpu.sync_copy(dst_hbm.at[row0], dst_i.at[0])
    _issue_g(0, 0, 0)
    _issue_i(1, 1, 1)

    def _quad(q, carry):
        for k in range(4):
            g = 4 * q + k
            p = k % 2

            @pl.when(g >= 2)
            def _():
                _drain_sx(p)

            @pl.when(g + 2 < NB1)
            def _():
                _issue_i(g + 2, (k + 2) % 4, p)

            _drain_g(p, k)

            def _edge(e):
                v = g1[p][e, :] + g2[p][e, :]
                v = jnp.maximum(v, 0.2 * v)
                exb[p][e, :] = jnp.exp(v)

            plsc.parallel_loop(0, B, unroll=4)(_edge)
            pltpu.async_copy(exb[p], den_sh.at[dst_i.at[k]],
                             sems[p], add=True)
            pltpu.async_copy(
                exb[p], ex_hbm.at[pl.ds((row0 + g) * B, B)], semx[p])

            @pl.when(g + 1 < NB1)
            def _():
                _drain_i(1 - p, (k + 1) % 4)
                _issue_g(g + 1, 1 - p, (k + 1) % 4)
        return carry

    lax.fori_loop(0, NB1 // 4, _quad, 0)
    _drain_sx(0)
    _drain_sx(1)
    plsc.subcore_barrier()
    pltpu.sync_copy(den_sh.at[pl.ds(tid * RPT, RPT)],
                    denp_hbm.at[cid, pl.ds(tid * RPT, RPT)])


@functools.partial(
    pl.kernel,
    out_type=jax.ShapeDtypeStruct((HEADS, NP, CHC), jnp.float32),
    mesh=_mesh,
    scratch_types=[
        pltpu.VMEM((4, B), jnp.int32),
        pltpu.VMEM((4, B), jnp.int32),
        pltpu.VMEM((B, L), jnp.float32),
        pltpu.VMEM((B, L), jnp.float32),
        pltpu.VMEM((B, CHC), jnp.float32),
        pltpu.VMEM((B, CHC), jnp.float32),
        pltpu.VMEM((B, CHC), jnp.float32),
        pltpu.VMEM((B, CHC), jnp.float32),
        pltpu.VMEM((ZR, CHC), jnp.float32),
        pltpu.SemaphoreType.DMA,
        pltpu.SemaphoreType.DMA,
        pltpu.SemaphoreType.DMA,
        pltpu.SemaphoreType.DMA,
        pltpu.SemaphoreType.DMA,
        pltpu.SemaphoreType.DMA,
        pltpu.VMEM_SHARED((NP, CHC), jnp.float32),
    ],
    compiler_params=pltpu.CompilerParams(
        use_tc_tiling_on_sc=False, needs_layout_passes=False),
)
def _sc_pass2(src_hbm, dst_hbm, ex_hbm, hh0, hh1, hh2, hh3,
              out_hbm, src_i, dst_i, exb0, exb1, hg0, hg1,
              sb0, sb1, zb, semg0, semg1, sems0, sems1, semi0, semi1,
              acc_sh):
    cid = lax.axis_index("c")
    tid = lax.axis_index("s")
    exb = (exb0, exb1)
    hg = (hg0, hg1)
    sb = (sb0, sb1)
    semg = (semg0, semg1)
    sems = (sems0, sems1)
    semi = (semi0, semi1)

    def _zrow(i, carry):
        zb[i, pl.ds(0, L)] = jnp.zeros((L,), jnp.float32)
        zb[i, pl.ds(L, L)] = jnp.zeros((L,), jnp.float32)
        return carry

    lax.fori_loop(0, ZR, _zrow, 0)
    iota = lax.iota(jnp.int32, L)
    jfull = [jnp.full((L,), j, jnp.int32) for j in range(L)]

    def _sweep(hh_ref, slot):
        cslot = jnp.full((L,), slot, jnp.int32)

        def _zcopy(k, carry):
            pltpu.sync_copy(zb, acc_sh.at[pl.ds(tid * RPT + k * ZR, ZR)])
            return carry

        lax.fori_loop(0, RPT // ZR, _zcopy, 0)
        plsc.subcore_barrier()
        row0 = tid * NB2

        def _issue_g(g, p, k):
            pltpu.async_copy(
                ex_hbm.at[pl.ds((row0 + g) * B, B)], exb[p], semg[p])
            pltpu.async_copy(hh_ref.at[src_i.at[k]], hg[p], semg[p])

        def _drain_g(p, k):
            pltpu.make_async_copy(
                ex_hbm.at[pl.ds(row0 * B, B)], exb[p], semg[p]).wait()
            pltpu.make_async_copy(
                hh_ref.at[src_i.at[k]], hg[p], semg[p]).wait()

        def _drain_s(p):
            pltpu.make_async_copy(
                sb[p], acc_sh.at[dst_i.at[0]], sems[p]).wait()

        def _issue_i(g, k, p):
            pltpu.async_copy(src_hbm.at[row0 + g], src_i.at[k], semi[p])
            pltpu.async_copy(dst_hbm.at[row0 + g], dst_i.at[k], semi[p])

        def _drain_i(p, k):
            pltpu.make_async_copy(
                src_hbm.at[row0], src_i.at[k], semi[p]).wait()
            pltpu.make_async_copy(
                dst_hbm.at[row0], dst_i.at[k], semi[p]).wait()

        pltpu.sync_copy(src_hbm.at[row0], src_i.at[0])
        pltpu.sync_copy(dst_hbm.at[row0], dst_i.at[0])
        _issue_g(0, 0, 0)
        _issue_i(1, 1, 1)

        def _quad(q, carry):
            for k in range(4):
                g = 4 * q + k
                p = k % 2

                @pl.when(g >= 2)
                def _():
                    _drain_s(p)

                @pl.when(g + 2 < NB2)
                def _():
                    _issue_i(g + 2, (k + 2) % 4, p)

                _drain_g(p, k)

                def _grp(i):
                    ridx = i * L + iota
                    cv = plsc.load_gather(exb[p], [ridx, cslot])
                    for j in range(L):
                        e = i * L + j
                        cj = _lane_bcast(cv, jfull[j])
                        sb[p][e, pl.ds(0, L)] = hg[p][e, pl.ds(0, L)] * cj
                        sb[p][e, pl.ds(L, L)] = hg[p][e, pl.ds(L, L)] * cj

                plsc.parallel_loop(0, B // L, unroll=2)(_grp)
                pltpu.async_copy(sb[p], acc_sh.at[dst_i.at[k]],
                                 sems[p], add=True)

                @pl.when(g + 1 < NB2)
                def _():
                    _drain_i(1 - p, (k + 1) % 4)
                    _issue_g(g + 1, 1 - p, (k + 1) % 4)
            return carry

        lax.fori_loop(0, NB2 // 4, _quad, 0)
        _drain_s(0)
        _drain_s(1)
        plsc.subcore_barrier()
        pltpu.sync_copy(acc_sh.at[pl.ds(tid * RPT, RPT)],
                        out_hbm.at[slot, pl.ds(tid * RPT, RPT)])
        plsc.subcore_barrier()

    @pl.when(cid == 0)
    def _():
        _sweep(hh0, 0)
        _sweep(hh1, 1)

    @pl.when(cid == 1)
    def _():
        _sweep(hh2, 2)
        _sweep(hh3, 3)


def _attn_mat(a):
    m = jnp.zeros((HID, L), jnp.float32)
    for h in range(HEADS):
        m = m.at[h * CHC:(h + 1) * CHC, h].set(a[h])
    return m


def kernel(x, edge_index, W1e, b1e, W2e, b2e, Wc1, as1, ad1, bc1,
           Wc2, as2, ad2, bc2, Wd1, bd1, Wd2, bd2, Wr1, br1, Wr2, br2):
    x_pad = jnp.zeros((NP, 8), jnp.float32).at[:NN].set(x)
    loop_idx = jnp.arange(NN, dtype=jnp.int32)
    pad_idx = jnp.full((EP - EE - NN,), NN, jnp.int32)
    src = jnp.concatenate(
        [edge_index[0].astype(jnp.int32), loop_idx, pad_idx]).reshape(ER, B)
    dst = jnp.concatenate(
        [edge_index[1].astype(jnp.int32), loop_idx, pad_idx]).reshape(ER, B)

    h0, h1, h2, h3, ts, td = _enc(
        x_pad, W1e, b1e.reshape(1, HID), W2e, b2e.reshape(1, HID),
        Wc1, _attn_mat(as1), _attn_mat(ad1))
    ex1, denp1 = _sc_pass1(src, dst, ts, td)
    out1 = _sc_pass2(src, dst, ex1, h0, h1, h2, h3)

    h0, h1, h2, h3, ts, td = _mid(
        out1, denp1, bc1.reshape(HEADS, CHC), Wc2,
        _attn_mat(as2), _attn_mat(ad2))
    ex2, denp2 = _sc_pass1(src, dst, ts, td)
    out2 = _sc_pass2(src, dst, ex2, h0, h1, h2, h3)

    err, rep = _dec(
        out2, denp2, bc2.reshape(HEADS, CHC), Wd1, bd1.reshape(1, 64),
        Wd2, bd2.reshape(1, 4), Wr1, br1.reshape(1, 64),
        Wr2, br2.reshape(1, 1))
    return (err[:NN], rep[:NN])


# speedup vs baseline: 2.2882x; 1.0527x over previous
import functools

import jax
import jax.numpy as jnp
from jax import lax
from jax.experimental import pallas as pl
from jax.experimental.pallas import tpu as pltpu
from jax.experimental.pallas import tpu_sc as plsc

NN = 50000
EE = 800000
HEADS = 4
CHC = 32
HID = 128

L = 16
NC = 2
NS = 16
NW = NC * NS

NP = 50176
RB = 512
GRID = NP // RB
RPT = NP // NS

B = 128
EP = 851968
ER = EP // B
NB1 = EP // NW // B
NB2 = EP // NS // B
ZR = 98
ZR1 = 392
G1 = 4
G2 = 2

_mesh = plsc.VectorSubcoreMesh(
    core_axis_name="c", subcore_axis_name="s", num_cores=NC, num_subcores=NS)
def _lane_bcast(v, idx):
    return lax.gather(
        v, idx[:, None],
        dimension_numbers=lax.GatherDimensionNumbers(
            offset_dims=(), collapsed_slice_dims=(0,), start_index_map=(0,)),
        slice_sizes=(1,),
        mode=lax.GatherScatterMode.PROMISE_IN_BOUNDS)


def _enc_body(x_ref, w1_ref, b1_ref, w2_ref, b2_ref, wc_ref, as_ref, ad_ref,
              h0_ref, h1_ref, h2_ref, h3_ref, ts_ref, td_ref):
    h = jnp.dot(x_ref[...], w1_ref[...], preferred_element_type=jnp.float32)
    h = jnp.maximum(h + b1_ref[...], 0.0)
    h = jnp.dot(h, w2_ref[...], preferred_element_type=jnp.float32) + b2_ref[...]
    hh = jnp.dot(h, wc_ref[...], preferred_element_type=jnp.float32)
    ts_ref[...] = jnp.dot(hh, as_ref[...], preferred_element_type=jnp.float32)
    td_ref[...] = jnp.dot(hh, ad_ref[...], preferred_element_type=jnp.float32)
    h0_ref[...] = hh[:, 0 * CHC:1 * CHC]
    h1_ref[...] = hh[:, 1 * CHC:2 * CHC]
    h2_ref[...] = hh[:, 2 * CHC:3 * CHC]
    h3_ref[...] = hh[:, 3 * CHC:4 * CHC]


def _mid_body(o_ref, dp_ref, bc_ref, wc_ref, as_ref, ad_ref,
              h0_ref, h1_ref, h2_ref, h3_ref, ts_ref, td_ref):
    dn = 1.0 / (dp_ref[0] + dp_ref[1] + 1e-16)
    g = jnp.concatenate(
        [jnp.maximum(o_ref[h] * dn[:, h:h + 1] + bc_ref[h], 0.0)
         for h in range(HEADS)], axis=1)
    hh = jnp.dot(g, wc_ref[...], preferred_element_type=jnp.float32)
    ts_ref[...] = jnp.dot(hh, as_ref[...], preferred_element_type=jnp.float32)
    td_ref[...] = jnp.dot(hh, ad_ref[...], preferred_element_type=jnp.float32)
    h0_ref[...] = hh[:, 0 * CHC:1 * CHC]
    h1_ref[...] = hh[:, 1 * CHC:2 * CHC]
    h2_ref[...] = hh[:, 2 * CHC:3 * CHC]
    h3_ref[...] = hh[:, 3 * CHC:4 * CHC]


def _dec_body(o_ref, dp_ref, bc_ref, wd1_ref, bd1_ref, wd2_ref, bd2_ref,
              wr1_ref, br1_ref, wr2_ref, br2_ref, err_ref, rep_ref):
    dn = 1.0 / (dp_ref[0] + dp_ref[1] + 1e-16)
    h2 = jnp.concatenate(
        [o_ref[h] * dn[:, h:h + 1] + bc_ref[h] for h in range(HEADS)],
        axis=1)
    e = jnp.maximum(
        jnp.dot(h2, wd1_ref[...], preferred_element_type=jnp.float32)
        + bd1_ref[...], 0.0)
    err_ref[...] = jnp.dot(e, wd2_ref[...],
                           preferred_element_type=jnp.float32) + bd2_ref[...]
    r = jnp.maximum(
        jnp.dot(h2, wr1_ref[...], preferred_element_type=jnp.float32)
        + br1_ref[...], 0.0)
    rep_ref[...] = jnp.dot(r, wr2_ref[...],
                           preferred_element_type=jnp.float32) + br2_ref[...]


def _full(shape):
    nd = len(shape)
    return pl.BlockSpec(shape, lambda i, _nd=nd: (0,) * _nd)


def _enc(x_pad, w1, b1, w2, b2, wc, a_s, a_d):
    return pl.pallas_call(
        _enc_body,
        grid=(GRID,),
        in_specs=[
            pl.BlockSpec((RB, 8), lambda i: (i, 0)),
            _full((8, HID)), _full((1, HID)), _full((HID, HID)),
            _full((1, HID)), _full((HID, HID)), _full((HID, L)),
            _full((HID, L)),
        ],
        out_specs=[pl.BlockSpec((RB, CHC), lambda i: (i, 0))] * HEADS
        + [pl.BlockSpec((RB, L), lambda i: (i, 0))] * 2,
        out_shape=[jax.ShapeDtypeStruct((NP, CHC), jnp.float32)] * HEADS
        + [jax.ShapeDtypeStruct((NP, L), jnp.float32)] * 2,
    )(x_pad, w1, b1, w2, b2, wc, a_s, a_d)


def _mid(o, dn, bc, wc, a_s, a_d):
    return pl.pallas_call(
        _mid_body,
        grid=(GRID,),
        in_specs=[
            pl.BlockSpec((HEADS, RB, CHC), lambda i: (0, i, 0)),
            pl.BlockSpec((NC, RB, L), lambda i: (0, i, 0)),
            _full((HEADS, CHC)), _full((HID, HID)), _full((HID, L)),
            _full((HID, L)),
        ],
        out_specs=[pl.BlockSpec((RB, CHC), lambda i: (i, 0))] * HEADS
        + [pl.BlockSpec((RB, L), lambda i: (i, 0))] * 2,
        out_shape=[jax.ShapeDtypeStruct((NP, CHC), jnp.float32)] * HEADS
        + [jax.ShapeDtypeStruct((NP, L), jnp.float32)] * 2,
    )(o, dn, bc, wc, a_s, a_d)


def _dec(o, dn, bc, wd1, bd1, wd2, bd2, wr1, br1, wr2, br2):
    return pl.pallas_call(
        _dec_body,
        grid=(GRID,),
        in_specs=[
            pl.BlockSpec((HEADS, RB, CHC), lambda i: (0, i, 0)),
            pl.BlockSpec((NC, RB, L), lambda i: (0, i, 0)),
            _full((HEADS, CHC)), _full((HID, 64)), _full((1, 64)),
            _full((64, 4)), _full((1, 4)), _full((HID, 64)), _full((1, 64)),
            _full((64, 1)), _full((1, 1)),
        ],
        out_specs=[pl.BlockSpec((RB, 4), lambda i: (i, 0)),
                   pl.BlockSpec((RB, 1), lambda i: (i, 0))],
        out_shape=[jax.ShapeDtypeStruct((NP, 4), jnp.float32),
                   jax.ShapeDtypeStruct((NP, 1), jnp.float32)],
    )(o, dn, bc, wd1, bd1, wd2, bd2, wr1, br1, wr2, br2)


@functools.partial(
    pl.kernel,
    out_type=(jax.ShapeDtypeStruct((HEADS, EP), jnp.float32),
              jax.ShapeDtypeStruct((NC, NP, L), jnp.float32)),
    mesh=_mesh,
    scratch_types=[
        pltpu.VMEM((4, B), jnp.int32),
        pltpu.VMEM((4, B), jnp.int32),
        pltpu.VMEM((B, L), jnp.float32),
        pltpu.VMEM((B, L), jnp.float32),
        pltpu.VMEM((B, L), jnp.float32),
        pltpu.VMEM((B, L), jnp.float32),
        pltpu.VMEM((B, L), jnp.float32),
        pltpu.VMEM((B, L), jnp.float32),
        pltpu.VMEM((HEADS, B), jnp.float32),
        pltpu.VMEM((HEADS, B), jnp.float32),
        pltpu.VMEM((ZR1, L), jnp.float32),
        pltpu.SemaphoreType.DMA,
        pltpu.SemaphoreType.DMA,
        pltpu.SemaphoreType.DMA,
        pltpu.SemaphoreType.DMA,
        pltpu.SemaphoreType.DMA,
        pltpu.SemaphoreType.DMA,
        pltpu.SemaphoreType.DMA,
        pltpu.SemaphoreType.DMA,
        pltpu.VMEM_SHARED((NP, L), jnp.float32),
    ],
    compiler_params=pltpu.CompilerParams(
        use_tc_tiling_on_sc=False, needs_layout_passes=False),
)
def _sc_pass1(src_hbm, dst_hbm, ts_hbm, td_hbm, ex_hbm, denp_hbm,
              src_i, dst_i, g1a, g1b, g2a, g2b, exba, exbb, exha, exhb, zb,
              semg0, semg1, sems0, sems1, semx0, semx1, semi0, semi1,
              den_sh):
    cid = lax.axis_index("c")
    tid = lax.axis_index("s")
    wid = cid * NS + tid
    g1 = (g1a, g1b)
    g2 = (g2a, g2b)
    exb = (exba, exbb)
    exh = (exha, exhb)
    iota = lax.iota(jnp.int32, L)
    hfull = [jnp.full((L,), h, jnp.int32) for h in range(HEADS)]
    semg = (semg0, semg1)
    sems = (sems0, sems1)
    semx = (semx0, semx1)
    semi = (semi0, semi1)

    def _zrow(i, carry):
        zb[i, :] = jnp.zeros((L,), jnp.float32)
        return carry

    lax.fori_loop(0, ZR1, _zrow, 0)

    def _zcopy(k, carry):
        pltpu.sync_copy(zb, den_sh.at[pl.ds(tid * RPT + k * ZR1, ZR1)])
        return carry

    lax.fori_loop(0, RPT // ZR1, _zcopy, 0)
    plsc.subcore_barrier()

    row0 = wid * NB1

    def _issue_g(g, p, k):
        pltpu.async_copy(ts_hbm.at[src_i.at[k]], g1[p], semg[p])
        pltpu.async_copy(td_hbm.at[dst_i.at[k]], g2[p], semg[p])

    def _drain_g(p, k):
        pltpu.make_async_copy(
            ts_hbm.at[src_i.at[k]], g1[p], semg[p]).wait()
        pltpu.make_async_copy(
            td_hbm.at[dst_i.at[k]], g2[p], semg[p]).wait()

    def _drain_sx(p):
        pltpu.make_async_copy(
            exb[p], den_sh.at[dst_i.at[0]], sems[p]).wait()
        pltpu.make_async_copy(
            exh[p], ex_hbm.at[:, pl.ds(row0 * B, B)], semx[p]).wait()

    def _issue_i(g, k, p):
        pltpu.async_copy(src_hbm.at[row0 + g], src_i.at[k], semi[p])
        pltpu.async_copy(dst_hbm.at[row0 + g], dst_i.at[k], semi[p])

    def _drain_i(p, k):
        pltpu.make_async_copy(
            src_hbm.at[row0], src_i.at[k], semi[p]).wait()
        pltpu.make_async_copy(
            dst_hbm.at[row0], dst_i.at[k], semi[p]).wait()

    pltpu.sync_copy(src_hbm.at[row0], src_i.at[0])
    pltpu.sync_copy(dst_hbm.at[row0], dst_i.at[0])
    _issue_g(0, 0, 0)
    _issue_i(1, 1, 1)

    def _quad(q, carry):
        for k in range(4):
            g = 4 * q + k
            p = k % 2

            @pl.when(g >= 2)
            def _():
                _drain_sx(p)

            @pl.when(g + 2 < NB1)
            def _():
                _issue_i(g + 2, (k + 2) % 4, p)

            _drain_g(p, k)

            def _edge(e):
                v = g1[p][e, :] + g2[p][e, :]
                v = jnp.maximum(v, 0.2 * v)
                exb[p][e, :] = jnp.exp(v)

            plsc.parallel_loop(0, B, unroll=4)(_edge)

            def _tr(i):
                ridx = i * L + iota
                for h in range(HEADS):
                    vh = plsc.load_gather(exb[p], [ridx, hfull[h]])
                    exh[p][h, pl.ds(i * L, L)] = vh

            plsc.parallel_loop(0, B // L)(_tr)
            pltpu.async_copy(exb[p], den_sh.at[dst_i.at[k]],
                             sems[p], add=True)
            pltpu.async_copy(
                exh[p], ex_hbm.at[:, pl.ds((row0 + g) * B, B)], semx[p])

            @pl.when(g + 1 < NB1)
            def _():
                _drain_i(1 - p, (k + 1) % 4)
                _issue_g(g + 1, 1 - p, (k + 1) % 4)
        return carry

    lax.fori_loop(0, NB1 // 4, _quad, 0)
    _drain_sx(0)
    _drain_sx(1)
    plsc.subcore_barrier()
    pltpu.sync_copy(den_sh.at[pl.ds(tid * RPT, RPT)],
                    denp_hbm.at[cid, pl.ds(tid * RPT, RPT)])


@functools.partial(
    pl.kernel,
    out_type=jax.ShapeDtypeStruct((HEADS, NP, CHC), jnp.float32),
    mesh=_mesh,
    scratch_types=[
        pltpu.VMEM((4, B), jnp.int32),
        pltpu.VMEM((4, B), jnp.int32),
        pltpu.VMEM((B,), jnp.float32),
        pltpu.VMEM((B,), jnp.float32),
        pltpu.VMEM((B, CHC), jnp.float32),
        pltpu.VMEM((B, CHC), jnp.float32),
        pltpu.VMEM((B, CHC), jnp.float32),
        pltpu.VMEM((B, CHC), jnp.float32),
        pltpu.VMEM((ZR, CHC), jnp.float32),
        pltpu.SemaphoreType.DMA,
        pltpu.SemaphoreType.DMA,
        pltpu.SemaphoreType.DMA,
        pltpu.SemaphoreType.DMA,
        pltpu.SemaphoreType.DMA,
        pltpu.SemaphoreType.DMA,
        pltpu.VMEM_SHARED((NP, CHC), jnp.float32),
    ],
    compiler_params=pltpu.CompilerParams(
        use_tc_tiling_on_sc=False, needs_layout_passes=False),
)
def _sc_pass2(src_hbm, dst_hbm, ex_hbm, hh0, hh1, hh2, hh3,
              out_hbm, src_i, dst_i, exb0, exb1, hg0, hg1,
              sb0, sb1, zb, semg0, semg1, sems0, sems1, semi0, semi1,
              acc_sh):
    cid = lax.axis_index("c")
    tid = lax.axis_index("s")
    exb = (exb0, exb1)
    hg = (hg0, hg1)
    sb = (sb0, sb1)
    semg = (semg0, semg1)
    sems = (sems0, sems1)
    semi = (semi0, semi1)

    def _zrow(i, carry):
        zb[i, pl.ds(0, L)] = jnp.zeros((L,), jnp.float32)
        zb[i, pl.ds(L, L)] = jnp.zeros((L,), jnp.float32)
        return carry

    lax.fori_loop(0, ZR, _zrow, 0)
    jfull = [jnp.full((L,), j, jnp.int32) for j in range(L)]

    def _sweep(hh_ref, slot):
        def _zcopy(k, carry):
            pltpu.sync_copy(zb, acc_sh.at[pl.ds(tid * RPT + k * ZR, ZR)])
            return carry

        lax.fori_loop(0, RPT // ZR, _zcopy, 0)
        plsc.subcore_barrier()
        row0 = tid * NB2

        def _issue_g(g, p, k):
            pltpu.async_copy(
                ex_hbm.at[slot, pl.ds((row0 + g) * B, B)], exb[p], semg[p])
            pltpu.async_copy(hh_ref.at[src_i.at[k]], hg[p], semg[p])

        def _drain_g(p, k):
            pltpu.make_async_copy(
                ex_hbm.at[slot, pl.ds(row0 * B, B)], exb[p], semg[p]).wait()
            pltpu.make_async_copy(
                hh_ref.at[src_i.at[k]], hg[p], semg[p]).wait()

        def _drain_s(p):
            pltpu.make_async_copy(
                sb[p], acc_sh.at[dst_i.at[0]], sems[p]).wait()

        def _issue_i(g, k, p):
            pltpu.async_copy(src_hbm.at[row0 + g], src_i.at[k], semi[p])
            pltpu.async_copy(dst_hbm.at[row0 + g], dst_i.at[k], semi[p])

        def _drain_i(p, k):
            pltpu.make_async_copy(
                src_hbm.at[row0], src_i.at[k], semi[p]).wait()
            pltpu.make_async_copy(
                dst_hbm.at[row0], dst_i.at[k], semi[p]).wait()

        pltpu.sync_copy(src_hbm.at[row0], src_i.at[0])
        pltpu.sync_copy(dst_hbm.at[row0], dst_i.at[0])
        _issue_g(0, 0, 0)
        _issue_i(1, 1, 1)

        def _quad(q, carry):
            for k in range(4):
                g = 4 * q + k
                p = k % 2

                @pl.when(g >= 2)
                def _():
                    _drain_s(p)

                @pl.when(g + 2 < NB2)
                def _():
                    _issue_i(g + 2, (k + 2) % 4, p)

                _drain_g(p, k)

                def _grp(i):
                    cv = exb[p][pl.ds(i * L, L)]
                    for j in range(L):
                        e = i * L + j
                        cj = _lane_bcast(cv, jfull[j])
                        sb[p][e, pl.ds(0, L)] = hg[p][e, pl.ds(0, L)] * cj
                        sb[p][e, pl.ds(L, L)] = hg[p][e, pl.ds(L, L)] * cj

                plsc.parallel_loop(0, B // L, unroll=2)(_grp)
                pltpu.async_copy(sb[p], acc_sh.at[dst_i.at[k]],
                                 sems[p], add=True)

                @pl.when(g + 1 < NB2)
                def _():
                    _drain_i(1 - p, (k + 1) % 4)
                    _issue_g(g + 1, 1 - p, (k + 1) % 4)
            return carry

        lax.fori_loop(0, NB2 // 4, _quad, 0)
        _drain_s(0)
        _drain_s(1)
        plsc.subcore_barrier()
        pltpu.sync_copy(acc_sh.at[pl.ds(tid * RPT, RPT)],
                        out_hbm.at[slot, pl.ds(tid * RPT, RPT)])
        plsc.subcore_barrier()

    @pl.when(cid == 0)
    def _():
        _sweep(hh0, 0)
        _sweep(hh1, 1)

    @pl.when(cid == 1)
    def _():
        _sweep(hh2, 2)
        _sweep(hh3, 3)


def _attn_mat(a):
    m = jnp.zeros((HID, L), jnp.float32)
    for h in range(HEADS):
        m = m.at[h * CHC:(h + 1) * CHC, h].set(a[h])
    return m


def kernel(x, edge_index, W1e, b1e, W2e, b2e, Wc1, as1, ad1, bc1,
           Wc2, as2, ad2, bc2, Wd1, bd1, Wd2, bd2, Wr1, br1, Wr2, br2):
    x_pad = jnp.zeros((NP, 8), jnp.float32).at[:NN].set(x)
    loop_idx = jnp.arange(NN, dtype=jnp.int32)
    pad_idx = jnp.full((EP - EE - NN,), NN, jnp.int32)
    src = jnp.concatenate(
        [edge_index[0].astype(jnp.int32), loop_idx, pad_idx]).reshape(ER, B)
    dst = jnp.concatenate(
        [edge_index[1].astype(jnp.int32), loop_idx, pad_idx]).reshape(ER, B)

    h0, h1, h2, h3, ts, td = _enc(
        x_pad, W1e, b1e.reshape(1, HID), W2e, b2e.reshape(1, HID),
        Wc1, _attn_mat(as1), _attn_mat(ad1))
    ex1, denp1 = _sc_pass1(src, dst, ts, td)
    out1 = _sc_pass2(src, dst, ex1, h0, h1, h2, h3)

    h0, h1, h2, h3, ts, td = _mid(
        out1, denp1, bc1.reshape(HEADS, CHC), Wc2,
        _attn_mat(as2), _attn_mat(ad2))
    ex2, denp2 = _sc_pass1(src, dst, ts, td)
    out2 = _sc_pass2(src, dst, ex2, h0, h1, h2, h3)

    err, rep = _dec(
        out2, denp2, bc2.reshape(HEADS, CHC), Wd1, bd1.reshape(1, 64),
        Wd2, bd2.reshape(1, 4), Wr1, br1.reshape(1, 64),
        Wr2, br2.reshape(1, 1))
    return (err[:NN], rep[:NN])


# speedup vs baseline: 2.6592x; 1.1621x over previous
import functools

import jax
import jax.numpy as jnp
from jax import lax
from jax.experimental import pallas as pl
from jax.experimental.pallas import tpu as pltpu
from jax.experimental.pallas import tpu_sc as plsc

NN = 50000
EE = 800000
HEADS = 4
CHC = 32
HID = 128

L = 16
NC = 2
NS = 16
NW = NC * NS

NP = 50176
RB = 512
GRID = NP // RB
RPT = NP // NS

B = 128
EP = 851968
ER = EP // B
NB1 = EP // NW // B
NB2 = EP // NS // B
ZR = 98
ZR1 = 392
G1 = 4
G2 = 2

_mesh = plsc.VectorSubcoreMesh(
    core_axis_name="c", subcore_axis_name="s", num_cores=NC, num_subcores=NS)
def _lane_bcast(v, idx):
    return lax.gather(
        v, idx[:, None],
        dimension_numbers=lax.GatherDimensionNumbers(
            offset_dims=(), collapsed_slice_dims=(0,), start_index_map=(0,)),
        slice_sizes=(1,),
        mode=lax.GatherScatterMode.PROMISE_IN_BOUNDS)


def _enc_body(x_ref, w1_ref, b1_ref, w2_ref, b2_ref, wc_ref, as_ref, ad_ref,
              h0_ref, h1_ref, h2_ref, h3_ref, ts_ref, td_ref):
    h = jnp.dot(x_ref[...], w1_ref[...], preferred_element_type=jnp.float32)
    h = jnp.maximum(h + b1_ref[...], 0.0)
    h = jnp.dot(h, w2_ref[...], preferred_element_type=jnp.float32) + b2_ref[...]
    hh = jnp.dot(h, wc_ref[...], preferred_element_type=jnp.float32)
    ts_ref[...] = jnp.dot(hh, as_ref[...], preferred_element_type=jnp.float32)
    td_ref[...] = jnp.dot(hh, ad_ref[...], preferred_element_type=jnp.float32)
    h0_ref[...] = hh[:, 0 * CHC:1 * CHC]
    h1_ref[...] = hh[:, 1 * CHC:2 * CHC]
    h2_ref[...] = hh[:, 2 * CHC:3 * CHC]
    h3_ref[...] = hh[:, 3 * CHC:4 * CHC]


def _mid_body(o_ref, dp_ref, bc_ref, wc_ref, as_ref, ad_ref,
              h0_ref, h1_ref, h2_ref, h3_ref, ts_ref, td_ref):
    dn = 1.0 / (dp_ref[0] + dp_ref[1] + 1e-16)
    g = jnp.concatenate(
        [jnp.maximum(o_ref[h] * dn[:, h:h + 1] + bc_ref[h], 0.0)
         for h in range(HEADS)], axis=1)
    hh = jnp.dot(g, wc_ref[...], preferred_element_type=jnp.float32)
    ts_ref[...] = jnp.dot(hh, as_ref[...], preferred_element_type=jnp.float32)
    td_ref[...] = jnp.dot(hh, ad_ref[...], preferred_element_type=jnp.float32)
    h0_ref[...] = hh[:, 0 * CHC:1 * CHC]
    h1_ref[...] = hh[:, 1 * CHC:2 * CHC]
    h2_ref[...] = hh[:, 2 * CHC:3 * CHC]
    h3_ref[...] = hh[:, 3 * CHC:4 * CHC]


def _dec_body(o_ref, dp_ref, bc_ref, wd1_ref, bd1_ref, wd2_ref, bd2_ref,
              wr1_ref, br1_ref, wr2_ref, br2_ref, err_ref, rep_ref):
    dn = 1.0 / (dp_ref[0] + dp_ref[1] + 1e-16)
    h2 = jnp.concatenate(
        [o_ref[h] * dn[:, h:h + 1] + bc_ref[h] for h in range(HEADS)],
        axis=1)
    e = jnp.maximum(
        jnp.dot(h2, wd1_ref[...], preferred_element_type=jnp.float32)
        + bd1_ref[...], 0.0)
    err_ref[...] = jnp.dot(e, wd2_ref[...],
                           preferred_element_type=jnp.float32) + bd2_ref[...]
    r = jnp.maximum(
        jnp.dot(h2, wr1_ref[...], preferred_element_type=jnp.float32)
        + br1_ref[...], 0.0)
    rep_ref[...] = jnp.dot(r, wr2_ref[...],
                           preferred_element_type=jnp.float32) + br2_ref[...]


def _full(shape):
    nd = len(shape)
    return pl.BlockSpec(shape, lambda i, _nd=nd: (0,) * _nd)


def _enc(x_pad, w1, b1, w2, b2, wc, a_s, a_d):
    return pl.pallas_call(
        _enc_body,
        grid=(GRID,),
        in_specs=[
            pl.BlockSpec((RB, 8), lambda i: (i, 0)),
            _full((8, HID)), _full((1, HID)), _full((HID, HID)),
            _full((1, HID)), _full((HID, HID)), _full((HID, L)),
            _full((HID, L)),
        ],
        out_specs=[pl.BlockSpec((RB, CHC), lambda i: (i, 0))] * HEADS
        + [pl.BlockSpec((RB, L), lambda i: (i, 0))] * 2,
        out_shape=[jax.ShapeDtypeStruct((NP, CHC), jnp.float32)] * HEADS
        + [jax.ShapeDtypeStruct((NP, L), jnp.float32)] * 2,
    )(x_pad, w1, b1, w2, b2, wc, a_s, a_d)


def _mid(o, dn, bc, wc, a_s, a_d):
    return pl.pallas_call(
        _mid_body,
        grid=(GRID,),
        in_specs=[
            pl.BlockSpec((HEADS, RB, CHC), lambda i: (0, i, 0)),
            pl.BlockSpec((NC, RB, L), lambda i: (0, i, 0)),
            _full((HEADS, CHC)), _full((HID, HID)), _full((HID, L)),
            _full((HID, L)),
        ],
        out_specs=[pl.BlockSpec((RB, CHC), lambda i: (i, 0))] * HEADS
        + [pl.BlockSpec((RB, L), lambda i: (i, 0))] * 2,
        out_shape=[jax.ShapeDtypeStruct((NP, CHC), jnp.float32)] * HEADS
        + [jax.ShapeDtypeStruct((NP, L), jnp.float32)] * 2,
    )(o, dn, bc, wc, a_s, a_d)


def _dec(o, dn, bc, wd1, bd1, wd2, bd2, wr1, br1, wr2, br2):
    return pl.pallas_call(
        _dec_body,
        grid=(GRID,),
        in_specs=[
            pl.BlockSpec((HEADS, RB, CHC), lambda i: (0, i, 0)),
            pl.BlockSpec((NC, RB, L), lambda i: (0, i, 0)),
            _full((HEADS, CHC)), _full((HID, 64)), _full((1, 64)),
            _full((64, 4)), _full((1, 4)), _full((HID, 64)), _full((1, 64)),
            _full((64, 1)), _full((1, 1)),
        ],
        out_specs=[pl.BlockSpec((RB, 4), lambda i: (i, 0)),
                   pl.BlockSpec((RB, 1), lambda i: (i, 0))],
        out_shape=[jax.ShapeDtypeStruct((NP, 4), jnp.float32),
                   jax.ShapeDtypeStruct((NP, 1), jnp.float32)],
    )(o, dn, bc, wd1, bd1, wd2, bd2, wr1, br1, wr2, br2)


@functools.partial(
    pl.kernel,
    out_type=(jax.ShapeDtypeStruct((HEADS, EP), jnp.float32),
              jax.ShapeDtypeStruct((NC, NP, L), jnp.float32)),
    mesh=_mesh,
    scratch_types=[
        pltpu.VMEM((4, B), jnp.int32),
        pltpu.VMEM((4, B), jnp.int32),
        pltpu.VMEM((B, L), jnp.float32),
        pltpu.VMEM((B, L), jnp.float32),
        pltpu.VMEM((B, L), jnp.float32),
        pltpu.VMEM((B, L), jnp.float32),
        pltpu.VMEM((B, L), jnp.float32),
        pltpu.VMEM((B, L), jnp.float32),
        pltpu.VMEM((HEADS, B), jnp.float32),
        pltpu.VMEM((HEADS, B), jnp.float32),
        pltpu.VMEM((ZR1, L), jnp.float32),
        pltpu.SemaphoreType.DMA,
        pltpu.SemaphoreType.DMA,
        pltpu.SemaphoreType.DMA,
        pltpu.SemaphoreType.DMA,
        pltpu.SemaphoreType.DMA,
        pltpu.SemaphoreType.DMA,
        pltpu.SemaphoreType.DMA,
        pltpu.SemaphoreType.DMA,
        pltpu.VMEM_SHARED((NP, L), jnp.float32),
    ],
    compiler_params=pltpu.CompilerParams(
        use_tc_tiling_on_sc=False, needs_layout_passes=False),
)
def _sc_pass1(src_hbm, dst_hbm, ts_hbm, td_hbm, ex_hbm, denp_hbm,
              src_i, dst_i, g1a, g1b, g2a, g2b, exba, exbb, exha, exhb, zb,
              semg0, semg1, sems0, sems1, semx0, semx1, semi0, semi1,
              den_sh):
    cid = lax.axis_index("c")
    tid = lax.axis_index("s")
    wid = cid * NS + tid
    g1 = (g1a, g1b)
    g2 = (g2a, g2b)
    exb = (exba, exbb)
    exh = (exha, exhb)
    iota = lax.iota(jnp.int32, L)
    hfull = [jnp.full((L,), h, jnp.int32) for h in range(HEADS)]
    semg = (semg0, semg1)
    sems = (sems0, sems1)
    semx = (semx0, semx1)
    semi = (semi0, semi1)

    def _zrow(i, carry):
        zb[i, :] = jnp.zeros((L,), jnp.float32)
        return carry

    lax.fori_loop(0, ZR1, _zrow, 0)

    def _zcopy(k, carry):
        pltpu.sync_copy(zb, den_sh.at[pl.ds(tid * RPT + k * ZR1, ZR1)])
        return carry

    lax.fori_loop(0, RPT // ZR1, _zcopy, 0)
    plsc.subcore_barrier()

    row0 = wid * NB1

    def _issue_g(g, p, k):
        pltpu.async_copy(ts_hbm.at[src_i.at[k]], g1[p], semg[p])
        pltpu.async_copy(td_hbm.at[dst_i.at[k]], g2[p], semg[p])

    def _drain_g(p, k):
        pltpu.make_async_copy(
            ts_hbm.at[src_i.at[k]], g1[p], semg[p]).wait()
        pltpu.make_async_copy(
            td_hbm.at[dst_i.at[k]], g2[p], semg[p]).wait()

    def _drain_sx(p):
        pltpu.make_async_copy(
            exb[p], den_sh.at[dst_i.at[0]], sems[p]).wait()
        pltpu.make_async_copy(
            exh[p], ex_hbm.at[:, pl.ds(row0 * B, B)], semx[p]).wait()

    def _issue_i(g, k, p):
        pltpu.async_copy(src_hbm.at[row0 + g], src_i.at[k], semi[p])
        pltpu.async_copy(dst_hbm.at[row0 + g], dst_i.at[k], semi[p])

    def _drain_i(p, k):
        pltpu.make_async_copy(
            src_hbm.at[row0], src_i.at[k], semi[p]).wait()
        pltpu.make_async_copy(
            dst_hbm.at[row0], dst_i.at[k], semi[p]).wait()

    pltpu.sync_copy(src_hbm.at[row0], src_i.at[0])
    pltpu.sync_copy(dst_hbm.at[row0], dst_i.at[0])
    _issue_g(0, 0, 0)
    _issue_i(1, 1, 1)

    def _quad(q, carry):
        for k in range(4):
            g = 4 * q + k
            p = k % 2

            @pl.when(g >= 2)
            def _():
                _drain_sx(p)

            @pl.when(g + 2 < NB1)
            def _():
                _issue_i(g + 2, (k + 2) % 4, p)

            _drain_g(p, k)

            def _edge(e):
                v = g1[p][e, :] + g2[p][e, :]
                v = jnp.maximum(v, 0.2 * v)
                exb[p][e, :] = jnp.exp(v)

            plsc.parallel_loop(0, B, unroll=4)(_edge)

            def _tr(i):
                ridx = i * L + iota
                for h in range(HEADS):
                    vh = plsc.load_gather(exb[p], [ridx, hfull[h]])
                    exh[p][h, pl.ds(i * L, L)] = vh

            plsc.parallel_loop(0, B // L)(_tr)
            pltpu.async_copy(exb[p], den_sh.at[dst_i.at[k]],
                             sems[p], add=True)
            pltpu.async_copy(
                exh[p], ex_hbm.at[:, pl.ds((row0 + g) * B, B)], semx[p])

            @pl.when(g + 1 < NB1)
            def _():
                _drain_i(1 - p, (k + 1) % 4)
                _issue_g(g + 1, 1 - p, (k + 1) % 4)
        return carry

    lax.fori_loop(0, NB1 // 4, _quad, 0)
    _drain_sx(0)
    _drain_sx(1)
    plsc.subcore_barrier()
    pltpu.sync_copy(den_sh.at[pl.ds(tid * RPT, RPT)],
                    denp_hbm.at[cid, pl.ds(tid * RPT, RPT)])


@functools.partial(
    pl.kernel,
    out_type=jax.ShapeDtypeStruct((HEADS, NP, CHC), jnp.float32),
    mesh=_mesh,
    scratch_types=[
        pltpu.VMEM((4, B), jnp.int32),
        pltpu.VMEM((4, B), jnp.int32),
        pltpu.VMEM((B,), jnp.float32),
        pltpu.VMEM((B,), jnp.float32),
        pltpu.VMEM((B, CHC), jnp.float32),
        pltpu.VMEM((B, CHC), jnp.float32),
        pltpu.VMEM((B, CHC), jnp.float32),
        pltpu.VMEM((B, CHC), jnp.float32),
        pltpu.VMEM((ZR, CHC), jnp.float32),
        pltpu.SemaphoreType.DMA,
        pltpu.SemaphoreType.DMA,
        pltpu.SemaphoreType.DMA,
        pltpu.SemaphoreType.DMA,
        pltpu.SemaphoreType.DMA,
        pltpu.SemaphoreType.DMA,
        pltpu.VMEM_SHARED((NP, CHC), jnp.float32),
    ],
    compiler_params=pltpu.CompilerParams(
        use_tc_tiling_on_sc=False, needs_layout_passes=False),
)
def _sc_pass2(src_hbm, dst_hbm, ex_hbm, hh0, hh1, hh2, hh3,
              out_hbm, src_i, dst_i, exb0, exb1, hg0, hg1,
              sb0, sb1, zb, semg0, semg1, sems0, sems1, semi0, semi1,
              acc_sh):
    cid = lax.axis_index("c")
    tid = lax.axis_index("s")
    exb = (exb0, exb1)
    hg = (hg0, hg1)
    sb = (sb0, sb1)
    semg = (semg0, semg1)
    sems = (sems0, sems1)
    semi = (semi0, semi1)

    def _zrow(i, carry):
        zb[i, pl.ds(0, L)] = jnp.zeros((L,), jnp.float32)
        zb[i, pl.ds(L, L)] = jnp.zeros((L,), jnp.float32)
        return carry

    lax.fori_loop(0, ZR, _zrow, 0)
    jfull = [jnp.full((L,), j, jnp.int32) for j in range(L)]

    def _sweep(hh_ref, slot):
        def _zcopy(k, carry):
            pltpu.sync_copy(zb, acc_sh.at[pl.ds(tid * RPT + k * ZR, ZR)])
            return carry

        lax.fori_loop(0, RPT // ZR, _zcopy, 0)
        plsc.subcore_barrier()
        row0 = tid * NB2

        def _issue_g(g, p, k):
            pltpu.async_copy(
                ex_hbm.at[slot, pl.ds((row0 + g) * B, B)], exb[p], semg[p])
            pltpu.async_copy(hh_ref.at[src_i.at[k]], hg[p], semg[p])

        def _drain_g(p, k):
            pltpu.make_async_copy(
                ex_hbm.at[slot, pl.ds(row0 * B, B)], exb[p], semg[p]).wait()
            pltpu.make_async_copy(
                hh_ref.at[src_i.at[k]], hg[p], semg[p]).wait()

        def _drain_s(p):
            pltpu.make_async_copy(
                sb[p], acc_sh.at[dst_i.at[0]], sems[p]).wait()

        def _issue_i(g, k, p):
            pltpu.async_copy(src_hbm.at[row0 + g], src_i.at[k], semi[p])
            pltpu.async_copy(dst_hbm.at[row0 + g], dst_i.at[k], semi[p])

        def _drain_i(p, k):
            pltpu.make_async_copy(
                src_hbm.at[row0], src_i.at[k], semi[p]).wait()
            pltpu.make_async_copy(
                dst_hbm.at[row0], dst_i.at[k], semi[p]).wait()

        pltpu.sync_copy(src_hbm.at[row0], src_i.at[0])
        pltpu.sync_copy(dst_hbm.at[row0], dst_i.at[0])
        _issue_g(0, 0, 0)
        _issue_i(1, 1, 1)

        def _quad(q, carry):
            for k in range(4):
                g = 4 * q + k
                p = k % 2

                @pl.when(g >= 2)
                def _():
                    _drain_s(p)

                @pl.when(g + 2 < NB2)
                def _():
                    _issue_i(g + 2, (k + 2) % 4, p)

                _drain_g(p, k)

                def _grp(i):
                    cv = exb[p][pl.ds(i * L, L)]
                    for j in range(L):
                        e = i * L + j
                        cj = _lane_bcast(cv, jfull[j])
                        sb[p][e, pl.ds(0, L)] = hg[p][e, pl.ds(0, L)] * cj
                        sb[p][e, pl.ds(L, L)] = hg[p][e, pl.ds(L, L)] * cj

                plsc.parallel_loop(0, B // L, unroll=4)(_grp)
                pltpu.async_copy(sb[p], acc_sh.at[dst_i.at[k]],
                                 sems[p], add=True)

                @pl.when(g + 1 < NB2)
                def _():
                    _drain_i(1 - p, (k + 1) % 4)
                    _issue_g(g + 1, 1 - p, (k + 1) % 4)
            return carry

        lax.fori_loop(0, NB2 // 4, _quad, 0)
        _drain_s(0)
        _drain_s(1)
        plsc.subcore_barrier()
        pltpu.sync_copy(acc_sh.at[pl.ds(tid * RPT, RPT)],
                        out_hbm.at[slot, pl.ds(tid * RPT, RPT)])
        plsc.subcore_barrier()

    @pl.when(cid == 0)
    def _():
        _sweep(hh0, 0)
        _sweep(hh1, 1)

    @pl.when(cid == 1)
    def _():
        _sweep(hh2, 2)
        _sweep(hh3, 3)


def _attn_mat(a):
    m = jnp.zeros((HID, L), jnp.float32)
    for h in range(HEADS):
        m = m.at[h * CHC:(h + 1) * CHC, h].set(a[h])
    return m


def kernel(x, edge_index, W1e, b1e, W2e, b2e, Wc1, as1, ad1, bc1,
           Wc2, as2, ad2, bc2, Wd1, bd1, Wd2, bd2, Wr1, br1, Wr2, br2):
    x_pad = jnp.zeros((NP, 8), jnp.float32).at[:NN].set(x)
    loop_idx = jnp.arange(NN, dtype=jnp.int32)
    pad_idx = jnp.full((EP - EE - NN,), NN, jnp.int32)
    src = jnp.concatenate(
        [edge_index[0].astype(jnp.int32), loop_idx, pad_idx]).reshape(ER, B)
    dst = jnp.concatenate(
        [edge_index[1].astype(jnp.int32), loop_idx, pad_idx]).reshape(ER, B)

    h0, h1, h2, h3, ts, td = _enc(
        x_pad, W1e, b1e.reshape(1, HID), W2e, b2e.reshape(1, HID),
        Wc1, _attn_mat(as1), _attn_mat(ad1))
    ex1, denp1 = _sc_pass1(src, dst, ts, td)
    out1 = _sc_pass2(src, dst, ex1, h0, h1, h2, h3)

    h0, h1, h2, h3, ts, td = _mid(
        out1, denp1, bc1.reshape(HEADS, CHC), Wc2,
        _attn_mat(as2), _attn_mat(ad2))
    ex2, denp2 = _sc_pass1(src, dst, ts, td)
    out2 = _sc_pass2(src, dst, ex2, h0, h1, h2, h3)

    err, rep = _dec(
        out2, denp2, bc2.reshape(HEADS, CHC), Wd1, bd1.reshape(1, 64),
        Wd2, bd2.reshape(1, 4), Wr1, br1.reshape(1, 64),
        Wr2, br2.reshape(1, 1))
    return (err[:NN], rep[:NN])
